# Initial kernel scaffold; baseline (speedup 1.0000x reference)
import jax
import jax.numpy as jnp
from jax.experimental import pallas as pl


def kernel(x, W_enc, b_enc, W_dec, b_dec):
    raise NotImplementedError("write your pallas kernel here")


# speedup vs baseline: 1.5812x; 1.5812x over previous
import functools

import jax
import jax.numpy as jnp
from jax import lax
from jax.experimental import pallas as pl
from jax.experimental.pallas import tpu as pltpu

KTOP = 32


def _matmul_body(x_ref, w_ref, b_ref, out_ref):
    acc = lax.dot_general(x_ref[...], w_ref[...], (((1,), (1,)), ((), ())),
                          preferred_element_type=jnp.float32)
    out_ref[...] = acc + b_ref[...]


def _encode(x, W_enc, b_enc, bt, bs, interpret=False):
    n, dm = x.shape
    ds = W_enc.shape[0]
    grid = (ds // bs, n // bt)
    return pl.pallas_call(
        _matmul_body,
        grid=grid,
        in_specs=[
            pl.BlockSpec((bt, dm), lambda j, i: (i, 0)),
            pl.BlockSpec((bs, dm), lambda j, i: (j, 0)),
            pl.BlockSpec((1, bs), lambda j, i: (0, j)),
        ],
        out_specs=pl.BlockSpec((bt, bs), lambda j, i: (i, j)),
        out_shape=jax.ShapeDtypeStruct((n, ds), jnp.float32),
        interpret=interpret,
    )(x, W_enc, b_enc.reshape(1, ds))


def _topk_body(pre_ref, acts_ref, idx_ref, *, bt, ds):
    colios = lax.broadcasted_iota(jnp.int32, (bt, ds), 1)
    kiota = lax.broadcasted_iota(jnp.int32, (bt, KTOP), 1)
    s = pre_ref[...]

    def body(i, carry):
        pm, pi, vals, inds = carry
        eligible = (s < pm[:, None]) | ((s == pm[:, None]) & (colios > pi[:, None]))
        masked = jnp.where(eligible, s, -jnp.inf)
        m = jnp.max(masked, axis=1)
        idx = jnp.min(jnp.where(masked == m[:, None], colios, ds), axis=1)
        vals = jnp.where(kiota == i, m[:, None], vals)
        inds = jnp.where(kiota == i, idx[:, None], inds)
        return m, idx, vals, inds

    pm0 = jnp.full((bt,), jnp.inf, jnp.float32)
    pi0 = jnp.full((bt,), -1, jnp.int32)
    v0 = jnp.zeros((bt, KTOP), jnp.float32)
    i0 = jnp.zeros((bt, KTOP), jnp.int32)
    _, _, vals, inds = lax.fori_loop(0, KTOP, body, (pm0, pi0, v0, i0))
    acts_ref[...] = vals
    idx_ref[...] = inds


def _topk(pre, bt, interpret=False):
    n, ds = pre.shape
    grid = (n // bt,)
    return pl.pallas_call(
        functools.partial(_topk_body, bt=bt, ds=ds),
        grid=grid,
        in_specs=[pl.BlockSpec((bt, ds), lambda i: (i, 0))],
        out_specs=[
            pl.BlockSpec((bt, KTOP), lambda i: (i, 0)),
            pl.BlockSpec((bt, KTOP), lambda i: (i, 0)),
        ],
        out_shape=[
            jax.ShapeDtypeStruct((n, KTOP), jnp.float32),
            jax.ShapeDtypeStruct((n, KTOP), jnp.int32),
        ],
        interpret=interpret,
    )(pre)


def _run(x, W_enc, b_enc, W_dec, b_dec, interpret=False):
    n, dm = x.shape
    ds = W_enc.shape[0]
    bt_a = min(512, n)
    bs_a = min(3072, ds)
    bt_b = min(128, n)
    pre = _encode(x, W_enc, b_enc, bt_a, bs_a, interpret)
    acts, idx = _topk(pre, bt_b, interpret)
    recon = jnp.einsum("nk,nkd->nd", acts, W_dec.T[idx]) + b_dec
    return recon, acts, idx


def kernel(x, W_enc, b_enc, W_dec, b_dec):
    return _run(x, W_enc, b_enc, W_dec, b_dec)


# speedup vs baseline: 1.9767x; 1.2501x over previous
import functools

import jax
import jax.numpy as jnp
from jax import lax
from jax.experimental import pallas as pl
from jax.experimental.pallas import tpu as pltpu
from jax.experimental.pallas import tpu_sc as plsc

KTOP = 32


def _matmul_body(x_ref, w_ref, b_ref, out_ref):
    acc = lax.dot_general(x_ref[...], w_ref[...], (((1,), (1,)), ((), ())),
                          preferred_element_type=jnp.float32)
    out_ref[...] = acc + b_ref[...]


def _encode(x, W_enc, b_enc, bt, bs, interpret=False):
    n, dm = x.shape
    ds = W_enc.shape[0]
    grid = (ds // bs, n // bt)
    return pl.pallas_call(
        _matmul_body,
        grid=grid,
        in_specs=[
            pl.BlockSpec((bt, dm), lambda j, i: (i, 0)),
            pl.BlockSpec((bs, dm), lambda j, i: (j, 0)),
            pl.BlockSpec((1, bs), lambda j, i: (0, j)),
        ],
        out_specs=pl.BlockSpec((bt, bs), lambda j, i: (i, j)),
        out_shape=jax.ShapeDtypeStruct((n, ds), jnp.float32),
        interpret=interpret,
    )(x, W_enc, b_enc.reshape(1, ds))


def _topk_body(pre_ref, acts_ref, idx_ref, *, bt, ds):
    colios = lax.broadcasted_iota(jnp.int32, (bt, ds), 1)
    kiota = lax.broadcasted_iota(jnp.int32, (bt, KTOP), 1)
    s = pre_ref[...]

    def body(i, carry):
        pm, pi, vals, inds = carry
        eligible = (s < pm[:, None]) | ((s == pm[:, None]) & (colios > pi[:, None]))
        masked = jnp.where(eligible, s, -jnp.inf)
        m = jnp.max(masked, axis=1)
        idx = jnp.min(jnp.where(masked == m[:, None], colios, ds), axis=1)
        vals = jnp.where(kiota == i, m[:, None], vals)
        inds = jnp.where(kiota == i, idx[:, None], inds)
        return m, idx, vals, inds

    pm0 = jnp.full((bt,), jnp.inf, jnp.float32)
    pi0 = jnp.full((bt,), -1, jnp.int32)
    v0 = jnp.zeros((bt, KTOP), jnp.float32)
    i0 = jnp.zeros((bt, KTOP), jnp.int32)
    _, _, vals, inds = lax.fori_loop(0, KTOP, body, (pm0, pi0, v0, i0))
    acts_ref[...] = vals
    idx_ref[...] = inds


def _topk(pre, bt, interpret=False):
    n, ds = pre.shape
    grid = (n // bt,)
    return pl.pallas_call(
        functools.partial(_topk_body, bt=bt, ds=ds),
        grid=grid,
        in_specs=[pl.BlockSpec((bt, ds), lambda i: (i, 0))],
        out_specs=[
            pl.BlockSpec((bt, KTOP), lambda i: (i, 0)),
            pl.BlockSpec((bt, KTOP), lambda i: (i, 0)),
        ],
        out_shape=[
            jax.ShapeDtypeStruct((n, KTOP), jnp.float32),
            jax.ShapeDtypeStruct((n, KTOP), jnp.int32),
        ],
        interpret=interpret,
    )(pre)


def _sc_decode(W_dec_T, acts, idx, b_dec):
    n = acts.shape[0]
    dm = W_dec_T.shape[1]
    nv = dm // 16
    info = plsc.get_sparse_core_info()
    nw = info.num_cores * info.num_subcores
    tpw = n // nw
    cc = 16
    mesh = plsc.VectorSubcoreMesh(core_axis_name="c", subcore_axis_name="s")

    @functools.partial(
        pl.kernel,
        mesh=mesh,
        out_type=jax.ShapeDtypeStruct((n, dm), jnp.float32),
        scratch_types=[
            pltpu.VMEM((cc, KTOP), jnp.int32),
            pltpu.VMEM((cc * KTOP,), jnp.float32),
            pltpu.VMEM((KTOP, dm), jnp.float32),
            pltpu.VMEM((cc, dm), jnp.float32),
            pltpu.VMEM((dm,), jnp.float32),
            pltpu.SemaphoreType.DMA,
        ],
    )
    def dec(wdt_hbm, acts_hbm, idx_hbm, bd_hbm, out_hbm,
            idx_v, acts_v, rows_v, out_v, bias_v, sem):
        wid = lax.axis_index("s") * info.num_cores + lax.axis_index("c")
        base = wid * tpw
        pltpu.sync_copy(bd_hbm, bias_v)

        def chunk_body(ci, _):
            cbase = base + ci * cc
            pltpu.sync_copy(idx_hbm.at[pl.ds(cbase, cc)], idx_v)
            pltpu.sync_copy(acts_hbm.at[pl.ds(cbase * KTOP, cc * KTOP)], acts_v)

            def tok_body(t, _):
                pltpu.async_copy(wdt_hbm.at[idx_v.at[t]], rows_v, sem).wait()

                av0 = acts_v[pl.ds(t * KTOP, 16)]
                av1 = acts_v[pl.ds(t * KTOP + 16, 16)]
                accs = [bias_v[pl.ds(16 * v, 16)] for v in range(nv)]
                for j in range(KTOP):
                    a = av0[j] if j < 16 else av1[j - 16]
                    for v in range(nv):
                        accs[v] = accs[v] + a * rows_v[j, pl.ds(16 * v, 16)]
                for v in range(nv):
                    out_v[t, pl.ds(16 * v, 16)] = accs[v]
                return 0

            lax.fori_loop(0, cc, tok_body, 0)
            pltpu.sync_copy(out_v, out_hbm.at[pl.ds(cbase, cc)])
            return 0

        lax.fori_loop(0, tpw // cc, chunk_body, 0)

    return dec(W_dec_T, acts.reshape(n * KTOP), idx, b_dec)


def _run(x, W_enc, b_enc, W_dec, b_dec, interpret=False):
    n, dm = x.shape
    ds = W_enc.shape[0]
    bt_a = min(512, n)
    bs_a = min(3072, ds)
    bt_b = min(128, n)
    pre = _encode(x, W_enc, b_enc, bt_a, bs_a, interpret)
    acts, idx = _topk(pre, bt_b, interpret)
    if interpret:
        recon = jnp.einsum("nk,nkd->nd", acts, W_dec.T[idx]) + b_dec
    else:
        recon = _sc_decode(jnp.transpose(W_dec), acts, idx, b_dec)
    return recon, acts, idx


def kernel(x, W_enc, b_enc, W_dec, b_dec):
    return _run(x, W_enc, b_enc, W_dec, b_dec)


# speedup vs baseline: 4.4573x; 2.2549x over previous
import functools

import jax
import jax.numpy as jnp
from jax import lax
from jax.experimental import pallas as pl
from jax.experimental.pallas import tpu as pltpu
from jax.experimental.pallas import tpu_sc as plsc

KTOP = 32


CHUNK = 128


def _matmul_body(x_ref, w_ref, b_ref, out_ref, cm_ref):
    acc = lax.dot_general(x_ref[...], w_ref[...], (((1,), (1,)), ((), ())),
                          preferred_element_type=jnp.float32)
    acc = acc + b_ref[...]
    out_ref[...] = acc
    bt, bs = acc.shape
    nck = bs // CHUNK
    cm_ref[...] = jnp.max(acc.reshape(bt, nck, CHUNK), axis=2).T


def _encode(x, W_enc, b_enc, bt, bs, interpret=False):
    n, dm = x.shape
    ds = W_enc.shape[0]
    grid = (ds // bs, n // bt)
    return pl.pallas_call(
        _matmul_body,
        grid=grid,
        in_specs=[
            pl.BlockSpec((bt, dm), lambda j, i: (i, 0)),
            pl.BlockSpec((bs, dm), lambda j, i: (j, 0)),
            pl.BlockSpec((1, bs), lambda j, i: (0, j)),
        ],
        out_specs=[
            pl.BlockSpec((bt, bs), lambda j, i: (i, j)),
            pl.BlockSpec((bs // CHUNK, bt), lambda j, i: (j, i)),
        ],
        out_shape=[
            jax.ShapeDtypeStruct((n, ds), jnp.float32),
            jax.ShapeDtypeStruct((ds // CHUNK, n), jnp.float32),
        ],
        interpret=interpret,
    )(x, W_enc, b_enc.reshape(1, ds))


def _topk_body(pre_ref, acts_ref, idx_ref, *, bt, ds):
    colios = lax.broadcasted_iota(jnp.int32, (bt, ds), 1)
    kiota = lax.broadcasted_iota(jnp.int32, (bt, KTOP), 1)
    s = pre_ref[...]

    def body(i, carry):
        pm, pi, vals, inds = carry
        eligible = (s < pm[:, None]) | ((s == pm[:, None]) & (colios > pi[:, None]))
        masked = jnp.where(eligible, s, -jnp.inf)
        m = jnp.max(masked, axis=1)
        idx = jnp.min(jnp.where(masked == m[:, None], colios, ds), axis=1)
        vals = jnp.where(kiota == i, m[:, None], vals)
        inds = jnp.where(kiota == i, idx[:, None], inds)
        return m, idx, vals, inds

    pm0 = jnp.full((bt,), jnp.inf, jnp.float32)
    pi0 = jnp.full((bt,), -1, jnp.int32)
    v0 = jnp.zeros((bt, KTOP), jnp.float32)
    i0 = jnp.zeros((bt, KTOP), jnp.int32)
    _, _, vals, inds = lax.fori_loop(0, KTOP, body, (pm0, pi0, v0, i0))
    acts_ref[...] = vals
    idx_ref[...] = inds


def _topk(pre, bt, interpret=False):
    n, ds = pre.shape
    grid = (n // bt,)
    return pl.pallas_call(
        functools.partial(_topk_body, bt=bt, ds=ds),
        grid=grid,
        in_specs=[pl.BlockSpec((bt, ds), lambda i: (i, 0))],
        out_specs=[
            pl.BlockSpec((bt, KTOP), lambda i: (i, 0)),
            pl.BlockSpec((bt, KTOP), lambda i: (i, 0)),
        ],
        out_shape=[
            jax.ShapeDtypeStruct((n, KTOP), jnp.float32),
            jax.ShapeDtypeStruct((n, KTOP), jnp.int32),
        ],
        interpret=interpret,
    )(pre)


def _chunksel_body(cm_ref, vals_ref, cids_ref, *, bt, nck):
    rowios = lax.broadcasted_iota(jnp.int32, (nck, bt), 0)
    kiota = lax.broadcasted_iota(jnp.int32, (bt, KTOP), 1)
    s = cm_ref[...]

    def body(i, carry):
        pm, pi, vals, inds = carry
        eligible = (s < pm[None, :]) | ((s == pm[None, :]) & (rowios > pi[None, :]))
        masked = jnp.where(eligible, s, -jnp.inf)
        m = jnp.max(masked, axis=0)
        idx = jnp.min(jnp.where(masked == m[None, :], rowios, nck), axis=0)
        vals = jnp.where(kiota == i, m[:, None], vals)
        inds = jnp.where(kiota == i, idx[:, None], inds)
        return m, idx, vals, inds

    pm0 = jnp.full((bt,), jnp.inf, jnp.float32)
    pi0 = jnp.full((bt,), -1, jnp.int32)
    v0 = jnp.zeros((bt, KTOP), jnp.float32)
    i0 = jnp.zeros((bt, KTOP), jnp.int32)
    _, _, vals, inds = lax.fori_loop(0, KTOP, body, (pm0, pi0, v0, i0))
    vals_ref[...] = vals
    cids_ref[...] = inds


def _chunksel(cmT, bt, interpret=False):
    nck, n = cmT.shape
    return pl.pallas_call(
        functools.partial(_chunksel_body, bt=bt, nck=nck),
        grid=(n // bt,),
        in_specs=[pl.BlockSpec((nck, bt), lambda i: (0, i))],
        out_specs=[
            pl.BlockSpec((bt, KTOP), lambda i: (i, 0)),
            pl.BlockSpec((bt, KTOP), lambda i: (i, 0)),
        ],
        out_shape=[
            jax.ShapeDtypeStruct((n, KTOP), jnp.float32),
            jax.ShapeDtypeStruct((n, KTOP), jnp.int32),
        ],
        interpret=interpret,
    )(cmT)


BIGI = 2 ** 30


def _cand_topk_body(v_ref, i_ref, acts_ref, idx_ref, *, bt):
    s = v_ref[...]
    gi = i_ref[...]
    kiota = lax.broadcasted_iota(jnp.int32, (bt, KTOP), 1)

    def body(i, carry):
        pm, pi, vals, inds = carry
        eligible = (s < pm[:, None]) | ((s == pm[:, None]) & (gi > pi[:, None]))
        masked = jnp.where(eligible, s, -jnp.inf)
        m = jnp.max(masked, axis=1)
        idx = jnp.min(jnp.where(masked == m[:, None], gi, BIGI), axis=1)
        vals = jnp.where(kiota == i, m[:, None], vals)
        inds = jnp.where(kiota == i, idx[:, None], inds)
        return m, idx, vals, inds

    pm0 = jnp.full((bt,), jnp.inf, jnp.float32)
    pi0 = jnp.full((bt,), -1, jnp.int32)
    v0 = jnp.zeros((bt, KTOP), jnp.float32)
    i0 = jnp.zeros((bt, KTOP), jnp.int32)
    _, _, vals, inds = lax.fori_loop(0, KTOP, body, (pm0, pi0, v0, i0))
    acts_ref[...] = vals
    idx_ref[...] = inds


def _cand_topk(cv, ci, bt, interpret=False):
    n, w = cv.shape
    return pl.pallas_call(
        functools.partial(_cand_topk_body, bt=bt),
        grid=(n // bt,),
        in_specs=[pl.BlockSpec((bt, w), lambda i: (i, 0)),
                  pl.BlockSpec((bt, w), lambda i: (i, 0))],
        out_specs=[pl.BlockSpec((bt, KTOP), lambda i: (i, 0)),
                   pl.BlockSpec((bt, KTOP), lambda i: (i, 0))],
        out_shape=[jax.ShapeDtypeStruct((n, KTOP), jnp.float32),
                   jax.ShapeDtypeStruct((n, KTOP), jnp.int32)],
        interpret=interpret,
    )(cv, ci)


CANDW = 512
CANDPAD = 544


def _sc_compact(pre2d, cids, cvals):
    n = cids.shape[0]
    nck = pre2d.shape[0] // n
    info = plsc.get_sparse_core_info()
    nw = info.num_cores * info.num_subcores
    tpw = n // nw
    cc = 8
    mesh = plsc.VectorSubcoreMesh(core_axis_name="c", subcore_axis_name="s")

    @functools.partial(
        pl.kernel,
        mesh=mesh,
        compiler_params=pltpu.CompilerParams(needs_layout_passes=False),
        out_type=[jax.ShapeDtypeStruct((n, CANDW), jnp.float32),
                  jax.ShapeDtypeStruct((n, CANDW), jnp.int32)],
        scratch_types=[
            pltpu.VMEM((cc, KTOP), jnp.int32),
            pltpu.VMEM((cc, KTOP), jnp.float32),
            pltpu.VMEM((16, CHUNK), jnp.float32),
            pltpu.VMEM((16, CHUNK), jnp.float32),
            pltpu.VMEM((CANDPAD,), jnp.float32),
            pltpu.VMEM((CANDPAD,), jnp.int32),
            pltpu.SemaphoreType.DMA,
        ],
    )
    def comp(pre_hbm, cids_hbm, cvals_hbm, ov_hbm, oi_hbm,
             cid_v, cv_v, g0_v, g1_v, vb_v, ib_v, sem):
        wid = lax.axis_index("s") * info.num_cores + lax.axis_index("c")
        base = wid * tpw
        lane = lax.iota(jnp.int32, 16)

        def chunk_body(ci, _):
            cbase = base + ci * cc
            pltpu.sync_copy(cids_hbm.at[pl.ds(cbase, cc)], cid_v)
            pltpu.sync_copy(cvals_hbm.at[pl.ds(cbase, cc)], cv_v)

            def tok_body(tl, _):
                t = cbase + tl
                c0 = cid_v[tl, pl.ds(0, 16)]
                c1 = cid_v[tl, pl.ds(16, 16)]
                cp0 = pltpu.async_copy(pre_hbm.at[c0 + t * nck], g0_v, sem)
                cp1 = pltpu.async_copy(pre_hbm.at[c1 + t * nck], g1_v, sem)
                cp0.wait()
                cp1.wait()
                t0 = cv_v[tl, pl.ds(16, 16)][15]
                for r in range(CANDPAD // 16):
                    vb_v[pl.ds(16 * r, 16)] = jnp.full((16,), -jnp.inf, jnp.float32)
                off = jnp.int32(0)
                for j in range(KTOP):
                    cvec = c0 if j < 16 else c1
                    buf = g0_v if j < 16 else g1_v
                    jj = j % 16
                    col0 = cvec[jj] * CHUNK
                    for r in range(CHUNK // 16):
                        v = buf[jj, pl.ds(16 * r, 16)]
                        m = v >= t0
                        iv = lane + (col0 + 16 * r)
                        inc = plsc.cumsum(jnp.where(m, 1, 0))
                        pos = inc + (off - 1)
                        plsc.store_scatter(vb_v, [pos], v, mask=m)
                        plsc.store_scatter(ib_v, [pos], iv, mask=m)
                        off = off + inc[15]
                    off = jnp.minimum(off, CANDW)
                pltpu.sync_copy(vb_v.at[pl.ds(0, CANDW)], ov_hbm.at[t])
                pltpu.sync_copy(ib_v.at[pl.ds(0, CANDW)], oi_hbm.at[t])
                return 0

            lax.fori_loop(0, cc, tok_body, 0)
            return 0

        lax.fori_loop(0, tpw // cc, chunk_body, 0)

    return comp(pre2d, cids, cvals)


def _sc_decode(W_dec_T, acts, idx, b_dec):
    n = acts.shape[0]
    dm = W_dec_T.shape[1]
    nv = dm // 16
    info = plsc.get_sparse_core_info()
    nw = info.num_cores * info.num_subcores
    tpw = n // nw
    cc = 16
    mesh = plsc.VectorSubcoreMesh(core_axis_name="c", subcore_axis_name="s")

    @functools.partial(
        pl.kernel,
        mesh=mesh,
        out_type=jax.ShapeDtypeStruct((n, dm), jnp.float32),
        scratch_types=[
            pltpu.VMEM((cc, KTOP), jnp.int32),
            pltpu.VMEM((cc * KTOP,), jnp.float32),
            pltpu.VMEM((KTOP, dm), jnp.float32),
            pltpu.VMEM((cc, dm), jnp.float32),
            pltpu.VMEM((dm,), jnp.float32),
            pltpu.SemaphoreType.DMA,
        ],
    )
    def dec(wdt_hbm, acts_hbm, idx_hbm, bd_hbm, out_hbm,
            idx_v, acts_v, rows_v, out_v, bias_v, sem):
        wid = lax.axis_index("s") * info.num_cores + lax.axis_index("c")
        base = wid * tpw
        pltpu.sync_copy(bd_hbm, bias_v)

        def chunk_body(ci, _):
            cbase = base + ci * cc
            pltpu.sync_copy(idx_hbm.at[pl.ds(cbase, cc)], idx_v)
            pltpu.sync_copy(acts_hbm.at[pl.ds(cbase * KTOP, cc * KTOP)], acts_v)

            def tok_body(t, _):
                pltpu.async_copy(wdt_hbm.at[idx_v.at[t]], rows_v, sem).wait()

                av0 = acts_v[pl.ds(t * KTOP, 16)]
                av1 = acts_v[pl.ds(t * KTOP + 16, 16)]
                accs = [bias_v[pl.ds(16 * v, 16)] for v in range(nv)]
                for j in range(KTOP):
                    a = av0[j] if j < 16 else av1[j - 16]
                    for v in range(nv):
                        accs[v] = accs[v] + a * rows_v[j, pl.ds(16 * v, 16)]
                for v in range(nv):
                    out_v[t, pl.ds(16 * v, 16)] = accs[v]
                return 0

            lax.fori_loop(0, cc, tok_body, 0)
            pltpu.sync_copy(out_v, out_hbm.at[pl.ds(cbase, cc)])
            return 0

        lax.fori_loop(0, tpw // cc, chunk_body, 0)

    return dec(W_dec_T, acts.reshape(n * KTOP), idx, b_dec)


def _run(x, W_enc, b_enc, W_dec, b_dec, interpret=False):
    n, dm = x.shape
    ds = W_enc.shape[0]
    bt_a = min(512, n)
    bs_a = min(3072, ds)
    bt_b = min(128, n)
    pre, cm = _encode(x, W_enc, b_enc, bt_a, bs_a, interpret)
    if interpret:
        acts, idx = _topk(pre, bt_b, interpret)
        recon = jnp.einsum("nk,nkd->nd", acts, W_dec.T[idx]) + b_dec
    else:
        cvals, cids = _chunksel(cm, 512)
        cv, cidx = _sc_compact(pre.reshape(n * (ds // CHUNK), CHUNK), cids, cvals)
        acts, idx = _cand_topk(cv, cidx, 256)
        recon = _sc_decode(jnp.transpose(W_dec), acts, idx, b_dec)
    return recon, acts, idx


def kernel(x, W_enc, b_enc, W_dec, b_dec):
    return _run(x, W_enc, b_enc, W_dec, b_dec)


# speedup vs baseline: 6.0841x; 1.3650x over previous
import functools

import jax
import jax.numpy as jnp
from jax import lax
from jax.experimental import pallas as pl
from jax.experimental.pallas import tpu as pltpu
from jax.experimental.pallas import tpu_sc as plsc

KTOP = 32


CHUNK = 128


def _matmul_body(x_ref, w_ref, b_ref, out_ref, cm_ref):
    acc = lax.dot_general(x_ref[...], w_ref[...], (((1,), (1,)), ((), ())),
                          preferred_element_type=jnp.float32)
    acc = acc + b_ref[...]
    out_ref[...] = acc
    bt, bs = acc.shape
    nck = bs // CHUNK
    cm_ref[...] = jnp.max(acc.reshape(bt, nck, CHUNK), axis=2).T


def _encode(x, W_enc, b_enc, bt, bs, interpret=False):
    n, dm = x.shape
    ds = W_enc.shape[0]
    grid = (ds // bs, n // bt)
    return pl.pallas_call(
        _matmul_body,
        grid=grid,
        in_specs=[
            pl.BlockSpec((bt, dm), lambda j, i: (i, 0)),
            pl.BlockSpec((bs, dm), lambda j, i: (j, 0)),
            pl.BlockSpec((1, bs), lambda j, i: (0, j)),
        ],
        out_specs=[
            pl.BlockSpec((bt, bs), lambda j, i: (i, j)),
            pl.BlockSpec((bs // CHUNK, bt), lambda j, i: (j, i)),
        ],
        out_shape=[
            jax.ShapeDtypeStruct((n, ds), jnp.float32),
            jax.ShapeDtypeStruct((ds // CHUNK, n), jnp.float32),
        ],
        interpret=interpret,
    )(x, W_enc, b_enc.reshape(1, ds))


def _topk_body(pre_ref, acts_ref, idx_ref, *, bt, ds):
    colios = lax.broadcasted_iota(jnp.int32, (bt, ds), 1)
    kiota = lax.broadcasted_iota(jnp.int32, (bt, KTOP), 1)
    s = pre_ref[...]

    def body(i, carry):
        pm, pi, vals, inds = carry
        eligible = (s < pm[:, None]) | ((s == pm[:, None]) & (colios > pi[:, None]))
        masked = jnp.where(eligible, s, -jnp.inf)
        m = jnp.max(masked, axis=1)
        idx = jnp.min(jnp.where(masked == m[:, None], colios, ds), axis=1)
        vals = jnp.where(kiota == i, m[:, None], vals)
        inds = jnp.where(kiota == i, idx[:, None], inds)
        return m, idx, vals, inds

    pm0 = jnp.full((bt,), jnp.inf, jnp.float32)
    pi0 = jnp.full((bt,), -1, jnp.int32)
    v0 = jnp.zeros((bt, KTOP), jnp.float32)
    i0 = jnp.zeros((bt, KTOP), jnp.int32)
    _, _, vals, inds = lax.fori_loop(0, KTOP, body, (pm0, pi0, v0, i0))
    acts_ref[...] = vals
    idx_ref[...] = inds


def _topk(pre, bt, interpret=False):
    n, ds = pre.shape
    grid = (n // bt,)
    return pl.pallas_call(
        functools.partial(_topk_body, bt=bt, ds=ds),
        grid=grid,
        in_specs=[pl.BlockSpec((bt, ds), lambda i: (i, 0))],
        out_specs=[
            pl.BlockSpec((bt, KTOP), lambda i: (i, 0)),
            pl.BlockSpec((bt, KTOP), lambda i: (i, 0)),
        ],
        out_shape=[
            jax.ShapeDtypeStruct((n, KTOP), jnp.float32),
            jax.ShapeDtypeStruct((n, KTOP), jnp.int32),
        ],
        interpret=interpret,
    )(pre)


def _chunksel_body(cm_ref, vals_ref, cids_ref, *, bt, nck):
    rowios = lax.broadcasted_iota(jnp.int32, (nck, bt), 0)
    kiota = lax.broadcasted_iota(jnp.int32, (bt, KTOP), 1)
    s = cm_ref[...]

    def body(i, carry):
        pm, pi, vals, inds = carry
        eligible = (s < pm[None, :]) | ((s == pm[None, :]) & (rowios > pi[None, :]))
        masked = jnp.where(eligible, s, -jnp.inf)
        m = jnp.max(masked, axis=0)
        idx = jnp.min(jnp.where(masked == m[None, :], rowios, nck), axis=0)
        vals = jnp.where(kiota == i, m[:, None], vals)
        inds = jnp.where(kiota == i, idx[:, None], inds)
        return m, idx, vals, inds

    pm0 = jnp.full((bt,), jnp.inf, jnp.float32)
    pi0 = jnp.full((bt,), -1, jnp.int32)
    v0 = jnp.zeros((bt, KTOP), jnp.float32)
    i0 = jnp.zeros((bt, KTOP), jnp.int32)
    _, _, vals, inds = lax.fori_loop(0, KTOP, body, (pm0, pi0, v0, i0))
    vals_ref[...] = vals
    cids_ref[...] = inds


def _chunksel(cmT, bt, interpret=False):
    nck, n = cmT.shape
    return pl.pallas_call(
        functools.partial(_chunksel_body, bt=bt, nck=nck),
        grid=(n // bt,),
        in_specs=[pl.BlockSpec((nck, bt), lambda i: (0, i))],
        out_specs=[
            pl.BlockSpec((bt, KTOP), lambda i: (i, 0)),
            pl.BlockSpec((bt, KTOP), lambda i: (i, 0)),
        ],
        out_shape=[
            jax.ShapeDtypeStruct((n, KTOP), jnp.float32),
            jax.ShapeDtypeStruct((n, KTOP), jnp.int32),
        ],
        interpret=interpret,
    )(cmT)


BIGI = 2 ** 30


def _cand_topk_body(v_ref, i_ref, acts_ref, idx_ref, *, bt):
    s = v_ref[...]
    gi = i_ref[...]
    kiota = lax.broadcasted_iota(jnp.int32, (bt, KTOP), 1)

    def body(i, carry):
        pm, pi, vals, inds = carry
        eligible = (s < pm[:, None]) | ((s == pm[:, None]) & (gi > pi[:, None]))
        masked = jnp.where(eligible, s, -jnp.inf)
        m = jnp.max(masked, axis=1)
        idx = jnp.min(jnp.where(masked == m[:, None], gi, BIGI), axis=1)
        vals = jnp.where(kiota == i, m[:, None], vals)
        inds = jnp.where(kiota == i, idx[:, None], inds)
        return m, idx, vals, inds

    pm0 = jnp.full((bt,), jnp.inf, jnp.float32)
    pi0 = jnp.full((bt,), -1, jnp.int32)
    v0 = jnp.zeros((bt, KTOP), jnp.float32)
    i0 = jnp.zeros((bt, KTOP), jnp.int32)
    _, _, vals, inds = lax.fori_loop(0, KTOP, body, (pm0, pi0, v0, i0))
    acts_ref[...] = vals
    idx_ref[...] = inds


def _cand_topk(cv, ci, bt, interpret=False):
    n, w = cv.shape
    return pl.pallas_call(
        functools.partial(_cand_topk_body, bt=bt),
        grid=(n // bt,),
        in_specs=[pl.BlockSpec((bt, w), lambda i: (i, 0)),
                  pl.BlockSpec((bt, w), lambda i: (i, 0))],
        out_specs=[pl.BlockSpec((bt, KTOP), lambda i: (i, 0)),
                   pl.BlockSpec((bt, KTOP), lambda i: (i, 0))],
        out_shape=[jax.ShapeDtypeStruct((n, KTOP), jnp.float32),
                   jax.ShapeDtypeStruct((n, KTOP), jnp.int32)],
        interpret=interpret,
    )(cv, ci)


CANDW = 512
CANDPAD = 544


def _sc_compact(pre2d, cids, cvals):
    n = cids.shape[0]
    nck = pre2d.shape[0] // n
    info = plsc.get_sparse_core_info()
    nw = info.num_cores * info.num_subcores
    tpw = n // nw
    cc = 32
    mesh = plsc.VectorSubcoreMesh(core_axis_name="c", subcore_axis_name="s")

    @functools.partial(
        pl.kernel,
        mesh=mesh,
        compiler_params=pltpu.CompilerParams(needs_layout_passes=False),
        out_type=[jax.ShapeDtypeStruct((n, CANDW), jnp.float32),
                  jax.ShapeDtypeStruct((n, CANDW), jnp.int32)],
        scratch_types=[
            pltpu.VMEM((cc, KTOP), jnp.int32),
            pltpu.VMEM((cc, KTOP), jnp.float32),
            pltpu.VMEM((16, CHUNK), jnp.float32),
            pltpu.VMEM((16, CHUNK), jnp.float32),
            pltpu.VMEM((16, CHUNK), jnp.float32),
            pltpu.VMEM((16, CHUNK), jnp.float32),
            pltpu.VMEM((CANDPAD,), jnp.float32),
            pltpu.VMEM((CANDPAD,), jnp.int32),
            pltpu.SemaphoreType.DMA,
            pltpu.SemaphoreType.DMA,
        ],
    )
    def comp(pre_hbm, cids_hbm, cvals_hbm, ov_hbm, oi_hbm,
             cid_v, cv_v, a0_v, a1_v, b0_v, b1_v, vb_v, ib_v, sema, semb):
        wid = lax.axis_index("s") * info.num_cores + lax.axis_index("c")
        base = wid * tpw
        lane = lax.iota(jnp.int32, 16)
        candw_v = jnp.full((16,), CANDW, jnp.int32)

        def issue(tl, t, d0, d1, sem):
            c0 = cid_v[tl, pl.ds(0, 16)] + t * nck
            c1 = cid_v[tl, pl.ds(16, 16)] + t * nck
            pltpu.async_copy(pre_hbm.at[c0], d0, sem)
            pltpu.async_copy(pre_hbm.at[c1], d1, sem)

        def drain(d0, d1, sem):
            pltpu.make_async_copy(pre_hbm.at[pl.ds(0, 16)], d0, sem).wait()
            pltpu.make_async_copy(pre_hbm.at[pl.ds(0, 16)], d1, sem).wait()

        def scan(tl, t, d0, d1):
            c0 = cid_v[tl, pl.ds(0, 16)]
            c1 = cid_v[tl, pl.ds(16, 16)]
            t0v = cv_v[tl, pl.ds(16, 16)]
            t0 = t0v[15]
            for r in range(CANDPAD // 16):
                vb_v[pl.ds(16 * r, 16)] = jnp.full((16,), -jnp.inf, jnp.float32)
            offv = jnp.zeros((16,), jnp.int32)
            for j in range(KTOP):
                cvec = c0 if j < 16 else c1
                buf = d0 if j < 16 else d1
                jj = j % 16
                col0 = cvec[jj] * CHUNK
                vs, ms = [], []
                for r in range(CHUNK // 16):
                    v = buf[jj, pl.ds(16 * r, 16)]
                    vs.append(v)
                    ms.append(v >= t0)
                incs = [plsc.cumsum(jnp.where(m, 1, 0)) for m in ms]
                cnts = [plsc.all_reduce_population_count(m) for m in ms]
                b = offv
                for r in range(CHUNK // 16):
                    pos = incs[r] + (b - 1)
                    iv = lane + (col0 + 16 * r)
                    plsc.store_scatter(vb_v, [pos], vs[r], mask=ms[r])
                    plsc.store_scatter(ib_v, [pos], iv, mask=ms[r])
                    b = jnp.minimum(b + cnts[r], candw_v)
                offv = b
            pltpu.sync_copy(vb_v.at[pl.ds(0, CANDW)], ov_hbm.at[t])
            pltpu.sync_copy(ib_v.at[pl.ds(0, CANDW)], oi_hbm.at[t])

        def chunk_body(ci, _):
            cbase = base + ci * cc
            pltpu.sync_copy(cids_hbm.at[pl.ds(cbase, cc)], cid_v)
            pltpu.sync_copy(cvals_hbm.at[pl.ds(cbase, cc)], cv_v)
            issue(0, cbase, a0_v, a1_v, sema)

            def pair_body(p, _):
                tla = 2 * p
                issue(tla + 1, cbase + tla + 1, b0_v, b1_v, semb)
                drain(a0_v, a1_v, sema)
                scan(tla, cbase + tla, a0_v, a1_v)

                @pl.when(p < cc // 2 - 1)
                def _():
                    issue(tla + 2, cbase + tla + 2, a0_v, a1_v, sema)

                drain(b0_v, b1_v, semb)
                scan(tla + 1, cbase + tla + 1, b0_v, b1_v)
                return 0

            lax.fori_loop(0, cc // 2, pair_body, 0)
            return 0

        lax.fori_loop(0, tpw // cc, chunk_body, 0)

    return comp(pre2d, cids, cvals)


def _sc_decode(W_dec_T, acts, idx, b_dec):
    n = acts.shape[0]
    dm = W_dec_T.shape[1]
    nv = dm // 16
    info = plsc.get_sparse_core_info()
    nw = info.num_cores * info.num_subcores
    tpw = n // nw
    cc = 16
    mesh = plsc.VectorSubcoreMesh(core_axis_name="c", subcore_axis_name="s")

    @functools.partial(
        pl.kernel,
        mesh=mesh,
        out_type=jax.ShapeDtypeStruct((n, dm), jnp.float32),
        scratch_types=[
            pltpu.VMEM((cc, KTOP), jnp.int32),
            pltpu.VMEM((cc * KTOP,), jnp.float32),
            pltpu.VMEM((KTOP, 16), jnp.float32),
            pltpu.VMEM((KTOP, dm), jnp.float32),
            pltpu.VMEM((KTOP, dm), jnp.float32),
            pltpu.VMEM((cc, dm), jnp.float32),
            pltpu.VMEM((dm,), jnp.float32),
            pltpu.SemaphoreType.DMA,
            pltpu.SemaphoreType.DMA,
        ],
    )
    def dec(wdt_hbm, acts_hbm, idx_hbm, bd_hbm, out_hbm,
            idx_v, acts_v, asp_v, rowsa_v, rowsb_v, out_v, bias_v, sema, semb):
        wid = lax.axis_index("s") * info.num_cores + lax.axis_index("c")
        base = wid * tpw
        pltpu.sync_copy(bd_hbm, bias_v)

        def compute(tl, rows):
            av0 = acts_v[pl.ds(tl * KTOP, 16)]
            av1 = acts_v[pl.ds(tl * KTOP + 16, 16)]
            for j in range(KTOP):
                a = av0[j] if j < 16 else av1[j - 16]
                asp_v[j, :] = jnp.full((16,), a, jnp.float32)
            for v in range(nv):
                out_v[tl, pl.ds(16 * v, 16)] = bias_v[pl.ds(16 * v, 16)]

            def jj_body(jj, _):
                a0 = asp_v[jj, :]
                a1 = asp_v[jj + 16, :]
                for v in range(nv):
                    plsc.addupdate(out_v.at[tl, pl.ds(16 * v, 16)],
                                   a0 * rows[jj, pl.ds(16 * v, 16)])
                for v in range(nv):
                    plsc.addupdate(out_v.at[tl, pl.ds(16 * v, 16)],
                                   a1 * rows[jj + 16, pl.ds(16 * v, 16)])
                return 0

            lax.fori_loop(0, 16, jj_body, 0)

        def drain(dst, sem):
            pltpu.make_async_copy(wdt_hbm.at[pl.ds(0, KTOP)], dst, sem).wait()

        def chunk_body(ci, _):
            cbase = base + ci * cc
            pltpu.sync_copy(idx_hbm.at[pl.ds(cbase, cc)], idx_v)
            pltpu.sync_copy(acts_hbm.at[pl.ds(cbase * KTOP, cc * KTOP)], acts_v)
            pltpu.async_copy(wdt_hbm.at[idx_v.at[0]], rowsa_v, sema)

            def pair_body(p, _):
                tla = 2 * p
                pltpu.async_copy(wdt_hbm.at[idx_v.at[tla + 1]], rowsb_v, semb)
                drain(rowsa_v, sema)
                compute(tla, rowsa_v)

                @pl.when(p < cc // 2 - 1)
                def _():
                    pltpu.async_copy(wdt_hbm.at[idx_v.at[tla + 2]], rowsa_v, sema)

                drain(rowsb_v, semb)
                compute(tla + 1, rowsb_v)
                return 0

            lax.fori_loop(0, cc // 2, pair_body, 0)
            pltpu.sync_copy(out_v, out_hbm.at[pl.ds(cbase, cc)])
            return 0

        lax.fori_loop(0, tpw // cc, chunk_body, 0)

    return dec(W_dec_T, acts.reshape(n * KTOP), idx, b_dec)


def _run(x, W_enc, b_enc, W_dec, b_dec, interpret=False):
    n, dm = x.shape
    ds = W_enc.shape[0]
    bt_a = min(512, n)
    bs_a = min(3072, ds)
    bt_b = min(128, n)
    pre, cm = _encode(x, W_enc, b_enc, bt_a, bs_a, interpret)
    if interpret:
        acts, idx = _topk(pre, bt_b, interpret)
        recon = jnp.einsum("nk,nkd->nd", acts, W_dec.T[idx]) + b_dec
    else:
        cvals, cids = _chunksel(cm, 512)
        cv, cidx = _sc_compact(pre.reshape(n * (ds // CHUNK), CHUNK), cids, cvals)
        acts, idx = _cand_topk(cv, cidx, 256)
        recon = _sc_decode(jnp.transpose(W_dec), acts, idx, b_dec)
    return recon, acts, idx


def kernel(x, W_enc, b_enc, W_dec, b_dec):
    return _run(x, W_enc, b_enc, W_dec, b_dec)


# speedup vs baseline: 9.3999x; 1.5450x over previous
import functools

import jax
import jax.numpy as jnp
from jax import lax
from jax.experimental import pallas as pl
from jax.experimental.pallas import tpu as pltpu
from jax.experimental.pallas import tpu_sc as plsc

KTOP = 32


CHUNK = 128


def _matmul_body(x_ref, w_ref, b_ref, out_ref, cm_ref):
    acc = lax.dot_general(x_ref[...], w_ref[...], (((1,), (1,)), ((), ())),
                          preferred_element_type=jnp.float32)
    acc = acc + b_ref[...]
    out_ref[...] = acc
    bt, bs = acc.shape
    nck = bs // CHUNK
    cm_ref[...] = jnp.max(acc.reshape(bt, nck, CHUNK), axis=2).T


def _encode(x, W_enc, b_enc, bt, bs, interpret=False):
    n, dm = x.shape
    ds = W_enc.shape[0]
    grid = (ds // bs, n // bt)
    return pl.pallas_call(
        _matmul_body,
        grid=grid,
        in_specs=[
            pl.BlockSpec((bt, dm), lambda j, i: (i, 0)),
            pl.BlockSpec((bs, dm), lambda j, i: (j, 0)),
            pl.BlockSpec((1, bs), lambda j, i: (0, j)),
        ],
        out_specs=[
            pl.BlockSpec((bt, bs), lambda j, i: (i, j)),
            pl.BlockSpec((bs // CHUNK, bt), lambda j, i: (j, i)),
        ],
        out_shape=[
            jax.ShapeDtypeStruct((n, ds), jnp.float32),
            jax.ShapeDtypeStruct((ds // CHUNK, n), jnp.float32),
        ],
        interpret=interpret,
    )(x, W_enc, b_enc.reshape(1, ds))


def _topk_body(pre_ref, acts_ref, idx_ref, *, bt, ds):
    colios = lax.broadcasted_iota(jnp.int32, (bt, ds), 1)
    kiota = lax.broadcasted_iota(jnp.int32, (bt, KTOP), 1)
    s = pre_ref[...]

    def body(i, carry):
        pm, pi, vals, inds = carry
        eligible = (s < pm[:, None]) | ((s == pm[:, None]) & (colios > pi[:, None]))
        masked = jnp.where(eligible, s, -jnp.inf)
        m = jnp.max(masked, axis=1)
        idx = jnp.min(jnp.where(masked == m[:, None], colios, ds), axis=1)
        vals = jnp.where(kiota == i, m[:, None], vals)
        inds = jnp.where(kiota == i, idx[:, None], inds)
        return m, idx, vals, inds

    pm0 = jnp.full((bt,), jnp.inf, jnp.float32)
    pi0 = jnp.full((bt,), -1, jnp.int32)
    v0 = jnp.zeros((bt, KTOP), jnp.float32)
    i0 = jnp.zeros((bt, KTOP), jnp.int32)
    _, _, vals, inds = lax.fori_loop(0, KTOP, body, (pm0, pi0, v0, i0))
    acts_ref[...] = vals
    idx_ref[...] = inds


def _topk(pre, bt, interpret=False):
    n, ds = pre.shape
    grid = (n // bt,)
    return pl.pallas_call(
        functools.partial(_topk_body, bt=bt, ds=ds),
        grid=grid,
        in_specs=[pl.BlockSpec((bt, ds), lambda i: (i, 0))],
        out_specs=[
            pl.BlockSpec((bt, KTOP), lambda i: (i, 0)),
            pl.BlockSpec((bt, KTOP), lambda i: (i, 0)),
        ],
        out_shape=[
            jax.ShapeDtypeStruct((n, KTOP), jnp.float32),
            jax.ShapeDtypeStruct((n, KTOP), jnp.int32),
        ],
        interpret=interpret,
    )(pre)


def _chunksel_body(cm_ref, vals_ref, cids_ref, *, bt, nck):
    rowios = lax.broadcasted_iota(jnp.int32, (nck, bt), 0)
    kiota = lax.broadcasted_iota(jnp.int32, (bt, KTOP), 1)
    s = cm_ref[...]

    def body(i, carry):
        pm, pi, vals, inds = carry
        eligible = (s < pm[None, :]) | ((s == pm[None, :]) & (rowios > pi[None, :]))
        masked = jnp.where(eligible, s, -jnp.inf)
        m = jnp.max(masked, axis=0)
        idx = jnp.min(jnp.where(masked == m[None, :], rowios, nck), axis=0)
        vals = jnp.where(kiota == i, m[:, None], vals)
        inds = jnp.where(kiota == i, idx[:, None], inds)
        return m, idx, vals, inds

    pm0 = jnp.full((bt,), jnp.inf, jnp.float32)
    pi0 = jnp.full((bt,), -1, jnp.int32)
    v0 = jnp.zeros((bt, KTOP), jnp.float32)
    i0 = jnp.zeros((bt, KTOP), jnp.int32)
    _, _, vals, inds = lax.fori_loop(0, KTOP, body, (pm0, pi0, v0, i0))
    vals_ref[...] = vals
    cids_ref[...] = inds


def _chunksel(cmT, bt, interpret=False):
    nck, n = cmT.shape
    return pl.pallas_call(
        functools.partial(_chunksel_body, bt=bt, nck=nck),
        grid=(n // bt,),
        in_specs=[pl.BlockSpec((nck, bt), lambda i: (0, i))],
        out_specs=[
            pl.BlockSpec((bt, KTOP), lambda i: (i, 0)),
            pl.BlockSpec((bt, KTOP), lambda i: (i, 0)),
        ],
        out_shape=[
            jax.ShapeDtypeStruct((n, KTOP), jnp.float32),
            jax.ShapeDtypeStruct((n, KTOP), jnp.int32),
        ],
        interpret=interpret,
    )(cmT)


BIGI = 2 ** 30


def _cand_topk_body(v_ref, i_ref, acts_ref, idx_ref, *, bt):
    s = v_ref[...]
    gi = i_ref[...]
    kiota = lax.broadcasted_iota(jnp.int32, (bt, KTOP), 1)

    def body(i, carry):
        pm, pi, vals, inds = carry
        eligible = (s < pm[:, None]) | ((s == pm[:, None]) & (gi > pi[:, None]))
        masked = jnp.where(eligible, s, -jnp.inf)
        m = jnp.max(masked, axis=1)
        idx = jnp.min(jnp.where(masked == m[:, None], gi, BIGI), axis=1)
        vals = jnp.where(kiota == i, m[:, None], vals)
        inds = jnp.where(kiota == i, idx[:, None], inds)
        return m, idx, vals, inds

    pm0 = jnp.full((bt,), jnp.inf, jnp.float32)
    pi0 = jnp.full((bt,), -1, jnp.int32)
    v0 = jnp.zeros((bt, KTOP), jnp.float32)
    i0 = jnp.zeros((bt, KTOP), jnp.int32)
    _, _, vals, inds = lax.fori_loop(0, KTOP, body, (pm0, pi0, v0, i0))
    acts_ref[...] = vals
    idx_ref[...] = inds


def _cand_topk(cv, ci, bt, interpret=False):
    n, w = cv.shape
    return pl.pallas_call(
        functools.partial(_cand_topk_body, bt=bt),
        grid=(n // bt,),
        in_specs=[pl.BlockSpec((bt, w), lambda i: (i, 0)),
                  pl.BlockSpec((bt, w), lambda i: (i, 0))],
        out_specs=[pl.BlockSpec((bt, KTOP), lambda i: (i, 0)),
                   pl.BlockSpec((bt, KTOP), lambda i: (i, 0))],
        out_shape=[jax.ShapeDtypeStruct((n, KTOP), jnp.float32),
                   jax.ShapeDtypeStruct((n, KTOP), jnp.int32)],
        interpret=interpret,
    )(cv, ci)


CANDW = 512
CANDPAD = 544


def _sc_compact(pre2d, cids, cvals):
    n = cids.shape[0]
    nck = pre2d.shape[0] // n
    info = plsc.get_sparse_core_info()
    nw = info.num_cores * info.num_subcores
    tpw = n // nw
    cc = 32
    mesh = plsc.VectorSubcoreMesh(core_axis_name="c", subcore_axis_name="s")

    @functools.partial(
        pl.kernel,
        mesh=mesh,
        compiler_params=pltpu.CompilerParams(needs_layout_passes=False),
        out_type=[jax.ShapeDtypeStruct((n, CANDW), jnp.float32),
                  jax.ShapeDtypeStruct((n, CANDW), jnp.int32)],
        scratch_types=[
            pltpu.VMEM((cc, KTOP), jnp.int32),
            pltpu.VMEM((cc, KTOP), jnp.float32),
            pltpu.VMEM((16, CHUNK), jnp.float32),
            pltpu.VMEM((16, CHUNK), jnp.float32),
            pltpu.VMEM((16, CHUNK), jnp.float32),
            pltpu.VMEM((16, CHUNK), jnp.float32),
            pltpu.VMEM((CANDPAD,), jnp.float32),
            pltpu.VMEM((CANDPAD,), jnp.int32),
            pltpu.SemaphoreType.DMA,
            pltpu.SemaphoreType.DMA,
        ],
    )
    def comp(pre_hbm, cids_hbm, cvals_hbm, ov_hbm, oi_hbm,
             cid_v, cv_v, a0_v, a1_v, b0_v, b1_v, vb_v, ib_v, sema, semb):
        wid = lax.axis_index("s") * info.num_cores + lax.axis_index("c")
        base = wid * tpw
        lane = lax.iota(jnp.int32, 16)
        candw_v = jnp.full((16,), CANDW, jnp.int32)

        def issue(tl, t, d0, d1, sem):
            c0 = cid_v[tl, pl.ds(0, 16)] + t * nck
            c1 = cid_v[tl, pl.ds(16, 16)] + t * nck
            pltpu.async_copy(pre_hbm.at[c0], d0, sem)
            pltpu.async_copy(pre_hbm.at[c1], d1, sem)

        def drain(d0, d1, sem):
            pltpu.make_async_copy(pre_hbm.at[pl.ds(0, 16)], d0, sem).wait()
            pltpu.make_async_copy(pre_hbm.at[pl.ds(0, 16)], d1, sem).wait()

        def scan(tl, t, d0, d1):
            c0 = cid_v[tl, pl.ds(0, 16)]
            c1 = cid_v[tl, pl.ds(16, 16)]
            t0v = cv_v[tl, pl.ds(16, 16)]
            t0 = t0v[15]
            for r in range(CANDPAD // 16):
                vb_v[pl.ds(16 * r, 16)] = jnp.full((16,), -jnp.inf, jnp.float32)
            offv = jnp.zeros((16,), jnp.int32)
            for j in range(KTOP):
                cvec = c0 if j < 16 else c1
                buf = d0 if j < 16 else d1
                jj = j % 16
                col0 = cvec[jj] * CHUNK
                vs, ms = [], []
                for r in range(CHUNK // 16):
                    v = buf[jj, pl.ds(16 * r, 16)]
                    vs.append(v)
                    ms.append(v >= t0)
                incs = [plsc.cumsum(jnp.where(m, 1, 0)) for m in ms]
                cnts = [plsc.all_reduce_population_count(m) for m in ms]
                b = offv
                for r in range(CHUNK // 16):
                    pos = incs[r] + (b - 1)
                    iv = lane + (col0 + 16 * r)
                    plsc.store_scatter(vb_v, [pos], vs[r], mask=ms[r])
                    plsc.store_scatter(ib_v, [pos], iv, mask=ms[r])
                    b = jnp.minimum(b + cnts[r], candw_v)
                offv = b
            pltpu.sync_copy(vb_v.at[pl.ds(0, CANDW)], ov_hbm.at[t])
            pltpu.sync_copy(ib_v.at[pl.ds(0, CANDW)], oi_hbm.at[t])

        def chunk_body(ci, _):
            cbase = base + ci * cc
            pltpu.sync_copy(cids_hbm.at[pl.ds(cbase, cc)], cid_v)
            pltpu.sync_copy(cvals_hbm.at[pl.ds(cbase, cc)], cv_v)
            issue(0, cbase, a0_v, a1_v, sema)

            def pair_body(p, _):
                tla = 2 * p
                issue(tla + 1, cbase + tla + 1, b0_v, b1_v, semb)
                drain(a0_v, a1_v, sema)
                scan(tla, cbase + tla, a0_v, a1_v)

                @pl.when(p < cc // 2 - 1)
                def _():
                    issue(tla + 2, cbase + tla + 2, a0_v, a1_v, sema)

                drain(b0_v, b1_v, semb)
                scan(tla + 1, cbase + tla + 1, b0_v, b1_v)
                return 0

            lax.fori_loop(0, cc // 2, pair_body, 0)
            return 0

        lax.fori_loop(0, tpw // cc, chunk_body, 0)

    return comp(pre2d, cids, cvals)


def _sc_decode(W_dec_T, acts, idx, b_dec):
    n = acts.shape[0]
    dm = W_dec_T.shape[1]
    nv = dm // 16
    info = plsc.get_sparse_core_info()
    nw = info.num_cores * info.num_subcores
    tpw = n // nw
    cc = 16
    mesh = plsc.VectorSubcoreMesh(core_axis_name="c", subcore_axis_name="s")

    @functools.partial(
        pl.kernel,
        mesh=mesh,
        out_type=jax.ShapeDtypeStruct((n, dm), jnp.float32),
        scratch_types=[
            pltpu.VMEM((cc, KTOP), jnp.int32),
            pltpu.VMEM((cc * KTOP,), jnp.float32),
            pltpu.VMEM((KTOP, 16), jnp.float32),
            pltpu.VMEM((KTOP, dm), jnp.float32),
            pltpu.VMEM((KTOP, dm), jnp.float32),
            pltpu.VMEM((cc, dm), jnp.float32),
            pltpu.VMEM((dm,), jnp.float32),
            pltpu.SemaphoreType.DMA,
            pltpu.SemaphoreType.DMA,
        ],
    )
    def dec(wdt_hbm, acts_hbm, idx_hbm, bd_hbm, out_hbm,
            idx_v, acts_v, asp_v, rowsa_v, rowsb_v, out_v, bias_v, sema, semb):
        wid = lax.axis_index("s") * info.num_cores + lax.axis_index("c")
        base = wid * tpw
        pltpu.sync_copy(bd_hbm, bias_v)

        def compute(tl, rows):
            av0 = acts_v[pl.ds(tl * KTOP, 16)]
            av1 = acts_v[pl.ds(tl * KTOP + 16, 16)]
            for j in range(KTOP):
                a = av0[j] if j < 16 else av1[j - 16]
                asp_v[j, :] = jnp.full((16,), a, jnp.float32)

            nh = nv // 2
            for half in range(2):
                base_v = half * nh

                def jj_body(jj, accs):
                    a0 = asp_v[jj, :]
                    a1 = asp_v[jj + 16, :]
                    return tuple(
                        accs[v]
                        + a0 * rows[jj, pl.ds(16 * (base_v + v), 16)]
                        + a1 * rows[jj + 16, pl.ds(16 * (base_v + v), 16)]
                        for v in range(nh))

                accs0 = tuple(bias_v[pl.ds(16 * (base_v + v), 16)]
                              for v in range(nh))
                accs = lax.fori_loop(0, 16, jj_body, accs0)
                for v in range(nh):
                    out_v[tl, pl.ds(16 * (base_v + v), 16)] = accs[v]

        def drain(dst, sem):
            pltpu.make_async_copy(wdt_hbm.at[pl.ds(0, KTOP)], dst, sem).wait()

        def chunk_body(ci, _):
            cbase = base + ci * cc
            pltpu.sync_copy(idx_hbm.at[pl.ds(cbase, cc)], idx_v)
            pltpu.sync_copy(acts_hbm.at[pl.ds(cbase * KTOP, cc * KTOP)], acts_v)
            pltpu.async_copy(wdt_hbm.at[idx_v.at[0]], rowsa_v, sema)

            def pair_body(p, _):
                tla = 2 * p
                pltpu.async_copy(wdt_hbm.at[idx_v.at[tla + 1]], rowsb_v, semb)
                drain(rowsa_v, sema)
                compute(tla, rowsa_v)

                @pl.when(p < cc // 2 - 1)
                def _():
                    pltpu.async_copy(wdt_hbm.at[idx_v.at[tla + 2]], rowsa_v, sema)

                drain(rowsb_v, semb)
                compute(tla + 1, rowsb_v)
                return 0

            lax.fori_loop(0, cc // 2, pair_body, 0)
            pltpu.sync_copy(out_v, out_hbm.at[pl.ds(cbase, cc)])
            return 0

        lax.fori_loop(0, tpw // cc, chunk_body, 0)

    return dec(W_dec_T, acts.reshape(n * KTOP), idx, b_dec)


def _run(x, W_enc, b_enc, W_dec, b_dec, interpret=False):
    n, dm = x.shape
    ds = W_enc.shape[0]
    bt_a = min(512, n)
    bs_a = min(3072, ds)
    bt_b = min(128, n)
    pre, cm = _encode(x, W_enc, b_enc, bt_a, bs_a, interpret)
    if interpret:
        acts, idx = _topk(pre, bt_b, interpret)
        recon = jnp.einsum("nk,nkd->nd", acts, W_dec.T[idx]) + b_dec
    else:
        cvals, cids = _chunksel(cm, 512)
        cv, cidx = _sc_compact(pre.reshape(n * (ds // CHUNK), CHUNK), cids, cvals)
        acts, idx = _cand_topk(cv, cidx, 256)
        recon = _sc_decode(jnp.transpose(W_dec), acts, idx, b_dec)
    return recon, acts, idx


def kernel(x, W_enc, b_enc, W_dec, b_dec):
    return _run(x, W_enc, b_enc, W_dec, b_dec)


# speedup vs baseline: 11.4041x; 1.2132x over previous
import functools

import jax
import jax.numpy as jnp
from jax import lax
from jax.experimental import pallas as pl
from jax.experimental.pallas import tpu as pltpu
from jax.experimental.pallas import tpu_sc as plsc

KTOP = 32


CHUNK = 128


def _matmul_body(x_ref, w_ref, b_ref, out_ref, cm_ref):
    acc = lax.dot_general(x_ref[...], w_ref[...], (((1,), (1,)), ((), ())),
                          preferred_element_type=jnp.float32)
    acc = acc + b_ref[...]
    out_ref[...] = acc
    bt, bs = acc.shape
    nck = bs // CHUNK
    cm_ref[...] = jnp.max(acc.reshape(bt, nck, CHUNK), axis=2).T


def _encode(x, W_enc, b_enc, bt, bs, interpret=False):
    n, dm = x.shape
    ds = W_enc.shape[0]
    grid = (ds // bs, n // bt)
    return pl.pallas_call(
        _matmul_body,
        grid=grid,
        in_specs=[
            pl.BlockSpec((bt, dm), lambda j, i: (i, 0)),
            pl.BlockSpec((bs, dm), lambda j, i: (j, 0)),
            pl.BlockSpec((1, bs), lambda j, i: (0, j)),
        ],
        out_specs=[
            pl.BlockSpec((bt, bs), lambda j, i: (i, j)),
            pl.BlockSpec((bs // CHUNK, bt), lambda j, i: (j, i)),
        ],
        out_shape=[
            jax.ShapeDtypeStruct((n, ds), jnp.float32),
            jax.ShapeDtypeStruct((ds // CHUNK, n), jnp.float32),
        ],
        interpret=interpret,
    )(x, W_enc, b_enc.reshape(1, ds))


def _topk_body(pre_ref, acts_ref, idx_ref, *, bt, ds):
    colios = lax.broadcasted_iota(jnp.int32, (bt, ds), 1)
    kiota = lax.broadcasted_iota(jnp.int32, (bt, KTOP), 1)
    s = pre_ref[...]

    def body(i, carry):
        pm, pi, vals, inds = carry
        eligible = (s < pm[:, None]) | ((s == pm[:, None]) & (colios > pi[:, None]))
        masked = jnp.where(eligible, s, -jnp.inf)
        m = jnp.max(masked, axis=1)
        idx = jnp.min(jnp.where(masked == m[:, None], colios, ds), axis=1)
        vals = jnp.where(kiota == i, m[:, None], vals)
        inds = jnp.where(kiota == i, idx[:, None], inds)
        return m, idx, vals, inds

    pm0 = jnp.full((bt,), jnp.inf, jnp.float32)
    pi0 = jnp.full((bt,), -1, jnp.int32)
    v0 = jnp.zeros((bt, KTOP), jnp.float32)
    i0 = jnp.zeros((bt, KTOP), jnp.int32)
    _, _, vals, inds = lax.fori_loop(0, KTOP, body, (pm0, pi0, v0, i0))
    acts_ref[...] = vals
    idx_ref[...] = inds


def _topk(pre, bt, interpret=False):
    n, ds = pre.shape
    grid = (n // bt,)
    return pl.pallas_call(
        functools.partial(_topk_body, bt=bt, ds=ds),
        grid=grid,
        in_specs=[pl.BlockSpec((bt, ds), lambda i: (i, 0))],
        out_specs=[
            pl.BlockSpec((bt, KTOP), lambda i: (i, 0)),
            pl.BlockSpec((bt, KTOP), lambda i: (i, 0)),
        ],
        out_shape=[
            jax.ShapeDtypeStruct((n, KTOP), jnp.float32),
            jax.ShapeDtypeStruct((n, KTOP), jnp.int32),
        ],
        interpret=interpret,
    )(pre)


def _chunksel_body(cm_ref, vals_ref, cids_ref, *, bt, nck):
    rowios = lax.broadcasted_iota(jnp.int32, (nck, bt), 0)
    kiota = lax.broadcasted_iota(jnp.int32, (bt, KTOP), 1)
    s = cm_ref[...]

    def body(i, carry):
        pm, pi, vals, inds = carry
        eligible = (s < pm[None, :]) | ((s == pm[None, :]) & (rowios > pi[None, :]))
        masked = jnp.where(eligible, s, -jnp.inf)
        m = jnp.max(masked, axis=0)
        idx = jnp.min(jnp.where(masked == m[None, :], rowios, nck), axis=0)
        vals = jnp.where(kiota == i, m[:, None], vals)
        inds = jnp.where(kiota == i, idx[:, None], inds)
        return m, idx, vals, inds

    pm0 = jnp.full((bt,), jnp.inf, jnp.float32)
    pi0 = jnp.full((bt,), -1, jnp.int32)
    v0 = jnp.zeros((bt, KTOP), jnp.float32)
    i0 = jnp.zeros((bt, KTOP), jnp.int32)
    _, _, vals, inds = lax.fori_loop(0, KTOP, body, (pm0, pi0, v0, i0))
    vals_ref[...] = vals
    cids_ref[...] = inds


def _chunksel(cmT, bt, n, tok0, interpret=False):
    nck = cmT.shape[0]
    off_b = tok0 // bt
    return pl.pallas_call(
        functools.partial(_chunksel_body, bt=bt, nck=nck),
        grid=(n // bt,),
        in_specs=[pl.BlockSpec((nck, bt), lambda i: (0, i + off_b))],
        out_specs=[
            pl.BlockSpec((bt, KTOP), lambda i: (i, 0)),
            pl.BlockSpec((bt, KTOP), lambda i: (i, 0)),
        ],
        out_shape=[
            jax.ShapeDtypeStruct((n, KTOP), jnp.float32),
            jax.ShapeDtypeStruct((n, KTOP), jnp.int32),
        ],
        interpret=interpret,
    )(cmT)


BIGI = 2 ** 30


def _cand_topk_body(v_ref, i_ref, acts_ref, idx_ref, *, bt):
    s = v_ref[...]
    gi = i_ref[...]
    kiota = lax.broadcasted_iota(jnp.int32, (bt, KTOP), 1)

    def body(i, carry):
        pm, pi, vals, inds = carry
        eligible = (s < pm[:, None]) | ((s == pm[:, None]) & (gi > pi[:, None]))
        masked = jnp.where(eligible, s, -jnp.inf)
        m = jnp.max(masked, axis=1)
        idx = jnp.min(jnp.where(masked == m[:, None], gi, BIGI), axis=1)
        vals = jnp.where(kiota == i, m[:, None], vals)
        inds = jnp.where(kiota == i, idx[:, None], inds)
        return m, idx, vals, inds

    pm0 = jnp.full((bt,), jnp.inf, jnp.float32)
    pi0 = jnp.full((bt,), -1, jnp.int32)
    v0 = jnp.zeros((bt, KTOP), jnp.float32)
    i0 = jnp.zeros((bt, KTOP), jnp.int32)
    _, _, vals, inds = lax.fori_loop(0, KTOP, body, (pm0, pi0, v0, i0))
    acts_ref[...] = vals
    idx_ref[...] = inds


def _cand_topk(cv, ci, bt, interpret=False):
    n, w = cv.shape
    return pl.pallas_call(
        functools.partial(_cand_topk_body, bt=bt),
        grid=(n // bt,),
        in_specs=[pl.BlockSpec((bt, w), lambda i: (i, 0)),
                  pl.BlockSpec((bt, w), lambda i: (i, 0))],
        out_specs=[pl.BlockSpec((bt, KTOP), lambda i: (i, 0)),
                   pl.BlockSpec((bt, KTOP), lambda i: (i, 0))],
        out_shape=[jax.ShapeDtypeStruct((n, KTOP), jnp.float32),
                   jax.ShapeDtypeStruct((n, KTOP), jnp.int32)],
        interpret=interpret,
    )(cv, ci)


CANDW = 512
CANDPAD = 544


def _sc_compact(pre2d, cids, cvals, ntot, tok0):
    n = cids.shape[0]
    nck = pre2d.shape[0] // ntot
    info = plsc.get_sparse_core_info()
    nw = info.num_cores * info.num_subcores
    tpw = n // nw
    cc = 32
    mesh = plsc.VectorSubcoreMesh(core_axis_name="c", subcore_axis_name="s")

    @functools.partial(
        pl.kernel,
        mesh=mesh,
        compiler_params=pltpu.CompilerParams(needs_layout_passes=False),
        out_type=[jax.ShapeDtypeStruct((n, CANDW), jnp.float32),
                  jax.ShapeDtypeStruct((n, CANDW), jnp.int32)],
        scratch_types=[
            pltpu.VMEM((cc, KTOP), jnp.int32),
            pltpu.VMEM((cc, KTOP), jnp.float32),
            pltpu.VMEM((16, CHUNK), jnp.float32),
            pltpu.VMEM((16, CHUNK), jnp.float32),
            pltpu.VMEM((16, CHUNK), jnp.float32),
            pltpu.VMEM((16, CHUNK), jnp.float32),
            pltpu.VMEM((CANDPAD,), jnp.float32),
            pltpu.VMEM((CANDPAD,), jnp.int32),
            pltpu.SemaphoreType.DMA,
            pltpu.SemaphoreType.DMA,
        ],
    )
    def comp(pre_hbm, cids_hbm, cvals_hbm, ov_hbm, oi_hbm,
             cid_v, cv_v, a0_v, a1_v, b0_v, b1_v, vb_v, ib_v, sema, semb):
        wid = lax.axis_index("s") * info.num_cores + lax.axis_index("c")
        base = wid * tpw
        lane = lax.iota(jnp.int32, 16)
        candw_v = jnp.full((16,), CANDW, jnp.int32)

        def issue(tl, t, d0, d1, sem):
            c0 = cid_v[tl, pl.ds(0, 16)] + (t + tok0) * nck
            c1 = cid_v[tl, pl.ds(16, 16)] + (t + tok0) * nck
            pltpu.async_copy(pre_hbm.at[c0], d0, sem)
            pltpu.async_copy(pre_hbm.at[c1], d1, sem)

        def drain(d0, d1, sem):
            pltpu.make_async_copy(pre_hbm.at[pl.ds(0, 16)], d0, sem).wait()
            pltpu.make_async_copy(pre_hbm.at[pl.ds(0, 16)], d1, sem).wait()

        def scan(tl, t, d0, d1):
            c0 = cid_v[tl, pl.ds(0, 16)]
            c1 = cid_v[tl, pl.ds(16, 16)]
            t0v = cv_v[tl, pl.ds(16, 16)]
            t0 = t0v[15]
            for r in range(CANDPAD // 16):
                vb_v[pl.ds(16 * r, 16)] = jnp.full((16,), -jnp.inf, jnp.float32)
            offv = jnp.zeros((16,), jnp.int32)
            for j in range(KTOP):
                cvec = c0 if j < 16 else c1
                buf = d0 if j < 16 else d1
                jj = j % 16
                col0 = cvec[jj] * CHUNK
                vs, ms = [], []
                for r in range(CHUNK // 16):
                    v = buf[jj, pl.ds(16 * r, 16)]
                    vs.append(v)
                    ms.append(v >= t0)
                incs = [plsc.cumsum(jnp.where(m, 1, 0)) for m in ms]
                cnts = [plsc.all_reduce_population_count(m) for m in ms]
                b = offv
                for r in range(CHUNK // 16):
                    pos = incs[r] + (b - 1)
                    iv = lane + (col0 + 16 * r)
                    plsc.store_scatter(vb_v, [pos], vs[r], mask=ms[r])
                    plsc.store_scatter(ib_v, [pos], iv, mask=ms[r])
                    b = jnp.minimum(b + cnts[r], candw_v)
                offv = b
            pltpu.sync_copy(vb_v.at[pl.ds(0, CANDW)], ov_hbm.at[t])
            pltpu.sync_copy(ib_v.at[pl.ds(0, CANDW)], oi_hbm.at[t])

        def chunk_body(ci, _):
            cbase = base + ci * cc
            pltpu.sync_copy(cids_hbm.at[pl.ds(cbase, cc)], cid_v)
            pltpu.sync_copy(cvals_hbm.at[pl.ds(cbase, cc)], cv_v)
            issue(0, cbase, a0_v, a1_v, sema)

            def pair_body(p, _):
                tla = 2 * p
                issue(tla + 1, cbase + tla + 1, b0_v, b1_v, semb)
                drain(a0_v, a1_v, sema)
                scan(tla, cbase + tla, a0_v, a1_v)

                @pl.when(p < cc // 2 - 1)
                def _():
                    issue(tla + 2, cbase + tla + 2, a0_v, a1_v, sema)

                drain(b0_v, b1_v, semb)
                scan(tla + 1, cbase + tla + 1, b0_v, b1_v)
                return 0

            lax.fori_loop(0, cc // 2, pair_body, 0)
            return 0

        lax.fori_loop(0, tpw // cc, chunk_body, 0)

    return comp(pre2d, cids, cvals)


def _sc_decode(W_dec_T, acts, idx, b_dec):
    n = acts.shape[0]
    dm = W_dec_T.shape[1]
    nv = dm // 16
    info = plsc.get_sparse_core_info()
    nw = info.num_cores * info.num_subcores
    tpw = n // nw
    cc = 16
    mesh = plsc.VectorSubcoreMesh(core_axis_name="c", subcore_axis_name="s")

    @functools.partial(
        pl.kernel,
        mesh=mesh,
        out_type=jax.ShapeDtypeStruct((n, dm), jnp.float32),
        scratch_types=[
            pltpu.VMEM((cc, KTOP), jnp.int32),
            pltpu.VMEM((cc * KTOP,), jnp.float32),
            pltpu.VMEM((KTOP, 16), jnp.float32),
            pltpu.VMEM((KTOP, dm), jnp.float32),
            pltpu.VMEM((KTOP, dm), jnp.float32),
            pltpu.VMEM((cc, dm), jnp.float32),
            pltpu.VMEM((dm,), jnp.float32),
            pltpu.SemaphoreType.DMA,
            pltpu.SemaphoreType.DMA,
        ],
    )
    def dec(wdt_hbm, acts_hbm, idx_hbm, bd_hbm, out_hbm,
            idx_v, acts_v, asp_v, rowsa_v, rowsb_v, out_v, bias_v, sema, semb):
        wid = lax.axis_index("s") * info.num_cores + lax.axis_index("c")
        base = wid * tpw
        pltpu.sync_copy(bd_hbm, bias_v)

        def compute(tl, rows):
            av0 = acts_v[pl.ds(tl * KTOP, 16)]
            av1 = acts_v[pl.ds(tl * KTOP + 16, 16)]
            for j in range(KTOP):
                a = av0[j] if j < 16 else av1[j - 16]
                asp_v[j, :] = jnp.full((16,), a, jnp.float32)

            nh = nv // 2
            for half in range(2):
                base_v = half * nh

                def jj_body(jj, accs):
                    a0 = asp_v[jj, :]
                    a1 = asp_v[jj + 16, :]
                    return tuple(
                        accs[v]
                        + a0 * rows[jj, pl.ds(16 * (base_v + v), 16)]
                        + a1 * rows[jj + 16, pl.ds(16 * (base_v + v), 16)]
                        for v in range(nh))

                accs0 = tuple(bias_v[pl.ds(16 * (base_v + v), 16)]
                              for v in range(nh))
                accs = lax.fori_loop(0, 16, jj_body, accs0)
                for v in range(nh):
                    out_v[tl, pl.ds(16 * (base_v + v), 16)] = accs[v]

        def drain(dst, sem):
            pltpu.make_async_copy(wdt_hbm.at[pl.ds(0, KTOP)], dst, sem).wait()

        def chunk_body(ci, _):
            cbase = base + ci * cc
            pltpu.sync_copy(idx_hbm.at[pl.ds(cbase, cc)], idx_v)
            pltpu.sync_copy(acts_hbm.at[pl.ds(cbase * KTOP, cc * KTOP)], acts_v)
            pltpu.async_copy(wdt_hbm.at[idx_v.at[0]], rowsa_v, sema)

            def pair_body(p, _):
                tla = 2 * p
                pltpu.async_copy(wdt_hbm.at[idx_v.at[tla + 1]], rowsb_v, semb)
                drain(rowsa_v, sema)
                compute(tla, rowsa_v)

                @pl.when(p < cc // 2 - 1)
                def _():
                    pltpu.async_copy(wdt_hbm.at[idx_v.at[tla + 2]], rowsa_v, sema)

                drain(rowsb_v, semb)
                compute(tla + 1, rowsb_v)
                return 0

            lax.fori_loop(0, cc // 2, pair_body, 0)
            pltpu.sync_copy(out_v, out_hbm.at[pl.ds(cbase, cc)])
            return 0

        lax.fori_loop(0, tpw // cc, chunk_body, 0)

    return dec(W_dec_T, acts.reshape(n * KTOP), idx, b_dec)


def _run(x, W_enc, b_enc, W_dec, b_dec, interpret=False):
    n, dm = x.shape
    ds = W_enc.shape[0]
    bt_a = min(512, n)
    bs_a = min(3072, ds)
    bt_b = min(128, n)
    pre, cm = _encode(x, W_enc, b_enc, bt_a, bs_a, interpret)
    if interpret:
        acts, idx = _topk(pre, bt_b, interpret)
        recon = jnp.einsum("nk,nkd->nd", acts, W_dec.T[idx]) + b_dec
    else:
        nslice = 2
        ns = n // nslice
        pre2d = pre.reshape(n * (ds // CHUNK), CHUNK)
        wdt = jnp.transpose(W_dec)
        parts = []
        for s in range(nslice):
            cvals, cids = _chunksel(cm, 512, ns, s * ns)
            cv, cidx = _sc_compact(pre2d, cids, cvals, n, s * ns)
            acts_s, idx_s = _cand_topk(cv, cidx, 256)
            recon_s = _sc_decode(wdt, acts_s, idx_s, b_dec)
            parts.append((recon_s, acts_s, idx_s))
        recon = jnp.concatenate([p[0] for p in parts], axis=0)
        acts = jnp.concatenate([p[1] for p in parts], axis=0)
        idx = jnp.concatenate([p[2] for p in parts], axis=0)
    return recon, acts, idx


def kernel(x, W_enc, b_enc, W_dec, b_dec):
    return _run(x, W_enc, b_enc, W_dec, b_dec)


# speedup vs baseline: 11.5572x; 1.0134x over previous
import functools

import jax
import jax.numpy as jnp
from jax import lax
from jax.experimental import pallas as pl
from jax.experimental.pallas import tpu as pltpu
from jax.experimental.pallas import tpu_sc as plsc

KTOP = 32


CHUNK = 128


def _matmul_body(x_ref, w_ref, b_ref, out_ref, cm_ref):
    acc = lax.dot_general(x_ref[...], w_ref[...], (((1,), (1,)), ((), ())),
                          preferred_element_type=jnp.float32)
    acc = acc + b_ref[...]
    out_ref[...] = acc
    bt, bs = acc.shape
    nck = bs // CHUNK
    cm_ref[...] = jnp.max(acc.reshape(bt, nck, CHUNK), axis=2).T


def _encode(x, W_enc, b_enc, bt, bs, interpret=False):
    n, dm = x.shape
    ds = W_enc.shape[0]
    grid = (ds // bs, n // bt)
    return pl.pallas_call(
        _matmul_body,
        grid=grid,
        in_specs=[
            pl.BlockSpec((bt, dm), lambda j, i: (i, 0)),
            pl.BlockSpec((bs, dm), lambda j, i: (j, 0)),
            pl.BlockSpec((1, bs), lambda j, i: (0, j)),
        ],
        out_specs=[
            pl.BlockSpec((bt, bs), lambda j, i: (i, j)),
            pl.BlockSpec((bs // CHUNK, bt), lambda j, i: (j, i)),
        ],
        out_shape=[
            jax.ShapeDtypeStruct((n, ds), jnp.float32),
            jax.ShapeDtypeStruct((ds // CHUNK, n), jnp.float32),
        ],
        interpret=interpret,
    )(x, W_enc, b_enc.reshape(1, ds))


def _topk_body(pre_ref, acts_ref, idx_ref, *, bt, ds):
    colios = lax.broadcasted_iota(jnp.int32, (bt, ds), 1)
    kiota = lax.broadcasted_iota(jnp.int32, (bt, KTOP), 1)
    s = pre_ref[...]

    def body(i, carry):
        pm, pi, vals, inds = carry
        eligible = (s < pm[:, None]) | ((s == pm[:, None]) & (colios > pi[:, None]))
        masked = jnp.where(eligible, s, -jnp.inf)
        m = jnp.max(masked, axis=1)
        idx = jnp.min(jnp.where(masked == m[:, None], colios, ds), axis=1)
        vals = jnp.where(kiota == i, m[:, None], vals)
        inds = jnp.where(kiota == i, idx[:, None], inds)
        return m, idx, vals, inds

    pm0 = jnp.full((bt,), jnp.inf, jnp.float32)
    pi0 = jnp.full((bt,), -1, jnp.int32)
    v0 = jnp.zeros((bt, KTOP), jnp.float32)
    i0 = jnp.zeros((bt, KTOP), jnp.int32)
    _, _, vals, inds = lax.fori_loop(0, KTOP, body, (pm0, pi0, v0, i0))
    acts_ref[...] = vals
    idx_ref[...] = inds


def _topk(pre, bt, interpret=False):
    n, ds = pre.shape
    grid = (n // bt,)
    return pl.pallas_call(
        functools.partial(_topk_body, bt=bt, ds=ds),
        grid=grid,
        in_specs=[pl.BlockSpec((bt, ds), lambda i: (i, 0))],
        out_specs=[
            pl.BlockSpec((bt, KTOP), lambda i: (i, 0)),
            pl.BlockSpec((bt, KTOP), lambda i: (i, 0)),
        ],
        out_shape=[
            jax.ShapeDtypeStruct((n, KTOP), jnp.float32),
            jax.ShapeDtypeStruct((n, KTOP), jnp.int32),
        ],
        interpret=interpret,
    )(pre)


def _chunksel_body(cm_ref, vals_ref, cids_ref, *, bt, nck):
    rowios = lax.broadcasted_iota(jnp.int32, (nck, bt), 0)
    kiota = lax.broadcasted_iota(jnp.int32, (bt, KTOP), 1)
    s = cm_ref[...]

    def body(i, carry):
        pm, pi, vals, inds = carry
        eligible = (s < pm[None, :]) | ((s == pm[None, :]) & (rowios > pi[None, :]))
        masked = jnp.where(eligible, s, -jnp.inf)
        m = jnp.max(masked, axis=0)
        idx = jnp.min(jnp.where(masked == m[None, :], rowios, nck), axis=0)
        vals = jnp.where(kiota == i, m[:, None], vals)
        inds = jnp.where(kiota == i, idx[:, None], inds)
        return m, idx, vals, inds

    pm0 = jnp.full((bt,), jnp.inf, jnp.float32)
    pi0 = jnp.full((bt,), -1, jnp.int32)
    v0 = jnp.zeros((bt, KTOP), jnp.float32)
    i0 = jnp.zeros((bt, KTOP), jnp.int32)
    _, _, vals, inds = lax.fori_loop(0, KTOP, body, (pm0, pi0, v0, i0))
    vals_ref[...] = vals
    cids_ref[...] = inds


def _chunksel(cmT, bt, n, tok0, interpret=False):
    nck = cmT.shape[0]
    off_b = tok0 // bt
    return pl.pallas_call(
        functools.partial(_chunksel_body, bt=bt, nck=nck),
        grid=(n // bt,),
        in_specs=[pl.BlockSpec((nck, bt), lambda i: (0, i + off_b))],
        out_specs=[
            pl.BlockSpec((bt, KTOP), lambda i: (i, 0)),
            pl.BlockSpec((bt, KTOP), lambda i: (i, 0)),
        ],
        out_shape=[
            jax.ShapeDtypeStruct((n, KTOP), jnp.float32),
            jax.ShapeDtypeStruct((n, KTOP), jnp.int32),
        ],
        interpret=interpret,
    )(cmT)


BIGI = 2 ** 30


def _cand_topk_body(v_ref, i_ref, acts_ref, idx_ref, *, bt):
    s = v_ref[...]
    gi = i_ref[...]
    kiota = lax.broadcasted_iota(jnp.int32, (bt, KTOP), 1)

    def body(i, carry):
        pm, pi, vals, inds = carry
        eligible = (s < pm[:, None]) | ((s == pm[:, None]) & (gi > pi[:, None]))
        masked = jnp.where(eligible, s, -jnp.inf)
        m = jnp.max(masked, axis=1)
        idx = jnp.min(jnp.where(masked == m[:, None], gi, BIGI), axis=1)
        vals = jnp.where(kiota == i, m[:, None], vals)
        inds = jnp.where(kiota == i, idx[:, None], inds)
        return m, idx, vals, inds

    pm0 = jnp.full((bt,), jnp.inf, jnp.float32)
    pi0 = jnp.full((bt,), -1, jnp.int32)
    v0 = jnp.zeros((bt, KTOP), jnp.float32)
    i0 = jnp.zeros((bt, KTOP), jnp.int32)
    _, _, vals, inds = lax.fori_loop(0, KTOP, body, (pm0, pi0, v0, i0))
    acts_ref[...] = vals
    idx_ref[...] = inds


def _cand_topk(cv, ci, bt, interpret=False):
    n, w = cv.shape
    return pl.pallas_call(
        functools.partial(_cand_topk_body, bt=bt),
        grid=(n // bt,),
        in_specs=[pl.BlockSpec((bt, w), lambda i: (i, 0)),
                  pl.BlockSpec((bt, w), lambda i: (i, 0))],
        out_specs=[pl.BlockSpec((bt, KTOP), lambda i: (i, 0)),
                   pl.BlockSpec((bt, KTOP), lambda i: (i, 0))],
        out_shape=[jax.ShapeDtypeStruct((n, KTOP), jnp.float32),
                   jax.ShapeDtypeStruct((n, KTOP), jnp.int32)],
        interpret=interpret,
    )(cv, ci)


CANDW = 512
CANDPAD = 544


def _sc_compact(pre2d, cids, cvals, ntot, tok0):
    n = cids.shape[0]
    nck = pre2d.shape[0] // ntot
    info = plsc.get_sparse_core_info()
    nw = info.num_cores * info.num_subcores
    tpw = n // nw
    cc = 32
    mesh = plsc.VectorSubcoreMesh(core_axis_name="c", subcore_axis_name="s")

    @functools.partial(
        pl.kernel,
        mesh=mesh,
        compiler_params=pltpu.CompilerParams(needs_layout_passes=False),
        out_type=[jax.ShapeDtypeStruct((n, CANDW), jnp.float32),
                  jax.ShapeDtypeStruct((n, CANDW), jnp.int32)],
        scratch_types=[
            pltpu.VMEM((cc, KTOP), jnp.int32),
            pltpu.VMEM((cc, KTOP), jnp.float32),
            pltpu.VMEM((16, CHUNK), jnp.float32),
            pltpu.VMEM((16, CHUNK), jnp.float32),
            pltpu.VMEM((16, CHUNK), jnp.float32),
            pltpu.VMEM((16, CHUNK), jnp.float32),
            pltpu.VMEM((CANDPAD,), jnp.float32),
            pltpu.VMEM((CANDPAD,), jnp.int32),
            pltpu.SemaphoreType.DMA,
            pltpu.SemaphoreType.DMA,
        ],
    )
    def comp(pre_hbm, cids_hbm, cvals_hbm, ov_hbm, oi_hbm,
             cid_v, cv_v, a0_v, a1_v, b0_v, b1_v, vb_v, ib_v, sema, semb):
        wid = lax.axis_index("s") * info.num_cores + lax.axis_index("c")
        base = wid * tpw
        lane = lax.iota(jnp.int32, 16)
        candw_v = jnp.full((16,), CANDW, jnp.int32)

        def issue(tl, t, d0, d1, sem):
            c0 = cid_v[tl, pl.ds(0, 16)] + (t + tok0) * nck
            c1 = cid_v[tl, pl.ds(16, 16)] + (t + tok0) * nck
            pltpu.async_copy(pre_hbm.at[c0], d0, sem)
            pltpu.async_copy(pre_hbm.at[c1], d1, sem)

        def drain(d0, d1, sem):
            pltpu.make_async_copy(pre_hbm.at[pl.ds(0, 16)], d0, sem).wait()
            pltpu.make_async_copy(pre_hbm.at[pl.ds(0, 16)], d1, sem).wait()

        def scan(tl, t, d0, d1):
            c0 = cid_v[tl, pl.ds(0, 16)]
            c1 = cid_v[tl, pl.ds(16, 16)]
            t0v = cv_v[tl, pl.ds(16, 16)]
            t0 = t0v[15]
            for r in range(CANDPAD // 16):
                vb_v[pl.ds(16 * r, 16)] = jnp.full((16,), -jnp.inf, jnp.float32)
            offv = jnp.zeros((16,), jnp.int32)
            for j in range(KTOP):
                cvec = c0 if j < 16 else c1
                buf = d0 if j < 16 else d1
                jj = j % 16
                col0 = cvec[jj] * CHUNK
                vs, ms = [], []
                for r in range(CHUNK // 16):
                    v = buf[jj, pl.ds(16 * r, 16)]
                    vs.append(v)
                    ms.append(v >= t0)
                incs = [plsc.cumsum(jnp.where(m, 1, 0)) for m in ms]
                cnts = [plsc.all_reduce_population_count(m) for m in ms]
                b = offv
                for r in range(CHUNK // 16):
                    pos = incs[r] + (b - 1)
                    iv = lane + (col0 + 16 * r)
                    plsc.store_scatter(vb_v, [pos], vs[r], mask=ms[r])
                    plsc.store_scatter(ib_v, [pos], iv, mask=ms[r])
                    b = jnp.minimum(b + cnts[r], candw_v)
                offv = b
            pltpu.sync_copy(vb_v.at[pl.ds(0, CANDW)], ov_hbm.at[t])
            pltpu.sync_copy(ib_v.at[pl.ds(0, CANDW)], oi_hbm.at[t])

        def chunk_body(ci, _):
            cbase = base + ci * cc
            pltpu.sync_copy(cids_hbm.at[pl.ds(cbase, cc)], cid_v)
            pltpu.sync_copy(cvals_hbm.at[pl.ds(cbase, cc)], cv_v)
            issue(0, cbase, a0_v, a1_v, sema)

            def pair_body(p, _):
                tla = 2 * p
                issue(tla + 1, cbase + tla + 1, b0_v, b1_v, semb)
                drain(a0_v, a1_v, sema)
                scan(tla, cbase + tla, a0_v, a1_v)

                @pl.when(p < cc // 2 - 1)
                def _():
                    issue(tla + 2, cbase + tla + 2, a0_v, a1_v, sema)

                drain(b0_v, b1_v, semb)
                scan(tla + 1, cbase + tla + 1, b0_v, b1_v)
                return 0

            lax.fori_loop(0, cc // 2, pair_body, 0)
            return 0

        lax.fori_loop(0, tpw // cc, chunk_body, 0)

    return comp(pre2d, cids, cvals)


def _sc_decode(W_dec_T, acts, idx, b_dec):
    n = acts.shape[0]
    dm = W_dec_T.shape[1]
    nv = dm // 16
    info = plsc.get_sparse_core_info()
    nw = info.num_cores * info.num_subcores
    tpw = n // nw
    cc = 16
    mesh = plsc.VectorSubcoreMesh(core_axis_name="c", subcore_axis_name="s")

    @functools.partial(
        pl.kernel,
        mesh=mesh,
        out_type=jax.ShapeDtypeStruct((n, dm), jnp.float32),
        scratch_types=[
            pltpu.VMEM((cc, KTOP), jnp.int32),
            pltpu.VMEM((cc * KTOP,), jnp.float32),
            pltpu.VMEM((KTOP, 16), jnp.float32),
            pltpu.VMEM((KTOP, dm), jnp.float32),
            pltpu.VMEM((KTOP, dm), jnp.float32),
            pltpu.VMEM((cc, dm), jnp.float32),
            pltpu.VMEM((dm,), jnp.float32),
            pltpu.SemaphoreType.DMA,
            pltpu.SemaphoreType.DMA,
        ],
    )
    def dec(wdt_hbm, acts_hbm, idx_hbm, bd_hbm, out_hbm,
            idx_v, acts_v, asp_v, rowsa_v, rowsb_v, out_v, bias_v, sema, semb):
        wid = lax.axis_index("s") * info.num_cores + lax.axis_index("c")
        base = wid * tpw
        pltpu.sync_copy(bd_hbm, bias_v)

        def compute(tl, rows):
            av0 = acts_v[pl.ds(tl * KTOP, 16)]
            av1 = acts_v[pl.ds(tl * KTOP + 16, 16)]
            for j in range(KTOP):
                a = av0[j] if j < 16 else av1[j - 16]
                asp_v[j, :] = jnp.full((16,), a, jnp.float32)

            nh = nv // 2
            for half in range(2):
                base_v = half * nh

                def jj_body(jj, accs):
                    a0 = asp_v[jj, :]
                    a1 = asp_v[jj + 16, :]
                    return tuple(
                        accs[v]
                        + a0 * rows[jj, pl.ds(16 * (base_v + v), 16)]
                        + a1 * rows[jj + 16, pl.ds(16 * (base_v + v), 16)]
                        for v in range(nh))

                accs0 = tuple(bias_v[pl.ds(16 * (base_v + v), 16)]
                              for v in range(nh))
                accs = lax.fori_loop(0, 16, jj_body, accs0)
                for v in range(nh):
                    out_v[tl, pl.ds(16 * (base_v + v), 16)] = accs[v]

        def drain(dst, sem):
            pltpu.make_async_copy(wdt_hbm.at[pl.ds(0, KTOP)], dst, sem).wait()

        def chunk_body(ci, _):
            cbase = base + ci * cc
            pltpu.sync_copy(idx_hbm.at[pl.ds(cbase, cc)], idx_v)
            pltpu.sync_copy(acts_hbm.at[pl.ds(cbase * KTOP, cc * KTOP)], acts_v)
            pltpu.async_copy(wdt_hbm.at[idx_v.at[0]], rowsa_v, sema)

            def pair_body(p, _):
                tla = 2 * p
                pltpu.async_copy(wdt_hbm.at[idx_v.at[tla + 1]], rowsb_v, semb)
                drain(rowsa_v, sema)
                compute(tla, rowsa_v)

                @pl.when(p < cc // 2 - 1)
                def _():
                    pltpu.async_copy(wdt_hbm.at[idx_v.at[tla + 2]], rowsa_v, sema)

                drain(rowsb_v, semb)
                compute(tla + 1, rowsb_v)
                return 0

            lax.fori_loop(0, cc // 2, pair_body, 0)
            pltpu.sync_copy(out_v, out_hbm.at[pl.ds(cbase, cc)])
            return 0

        lax.fori_loop(0, tpw // cc, chunk_body, 0)

    return dec(W_dec_T, acts.reshape(n * KTOP), idx, b_dec)


def _run(x, W_enc, b_enc, W_dec, b_dec, interpret=False):
    n, dm = x.shape
    ds = W_enc.shape[0]
    bt_a = min(512, n)
    bs_a = min(3072, ds)
    bt_b = min(128, n)
    pre, cm = _encode(x, W_enc, b_enc, bt_a, bs_a, interpret)
    if interpret:
        acts, idx = _topk(pre, bt_b, interpret)
        recon = jnp.einsum("nk,nkd->nd", acts, W_dec.T[idx]) + b_dec
    else:
        nslice = 4
        ns = n // nslice
        pre2d = pre.reshape(n * (ds // CHUNK), CHUNK)
        wdt = jnp.transpose(W_dec)
        parts = []
        for s in range(nslice):
            cvals, cids = _chunksel(cm, 512, ns, s * ns)
            cv, cidx = _sc_compact(pre2d, cids, cvals, n, s * ns)
            acts_s, idx_s = _cand_topk(cv, cidx, 256)
            recon_s = _sc_decode(wdt, acts_s, idx_s, b_dec)
            parts.append((recon_s, acts_s, idx_s))
        recon = jnp.concatenate([p[0] for p in parts], axis=0)
        acts = jnp.concatenate([p[1] for p in parts], axis=0)
        idx = jnp.concatenate([p[2] for p in parts], axis=0)
    return recon, acts, idx


def kernel(x, W_enc, b_enc, W_dec, b_dec):
    return _run(x, W_enc, b_enc, W_dec, b_dec)


# speedup vs baseline: 11.5658x; 1.0007x over previous
import functools

import jax
import jax.numpy as jnp
from jax import lax
from jax.experimental import pallas as pl
from jax.experimental.pallas import tpu as pltpu
from jax.experimental.pallas import tpu_sc as plsc

KTOP = 32


CHUNK = 128


def _matmul_body(x_ref, w_ref, b_ref, out_ref, cm_ref):
    acc = lax.dot_general(x_ref[...], w_ref[...], (((1,), (1,)), ((), ())),
                          preferred_element_type=jnp.float32)
    acc = acc + b_ref[...]
    out_ref[...] = acc
    bt, bs = acc.shape
    nck = bs // CHUNK
    cm_ref[...] = jnp.max(acc.reshape(bt, nck, CHUNK), axis=2).T


def _encode(x, W_enc, b_enc, bt, bs):
    n, dm = x.shape
    ds = W_enc.shape[0]
    grid = (ds // bs, n // bt)
    return pl.pallas_call(
        _matmul_body,
        grid=grid,
        in_specs=[
            pl.BlockSpec((bt, dm), lambda j, i: (i, 0)),
            pl.BlockSpec((bs, dm), lambda j, i: (j, 0)),
            pl.BlockSpec((1, bs), lambda j, i: (0, j)),
        ],
        out_specs=[
            pl.BlockSpec((bt, bs), lambda j, i: (i, j)),
            pl.BlockSpec((bs // CHUNK, bt), lambda j, i: (j, i)),
        ],
        out_shape=[
            jax.ShapeDtypeStruct((n, ds), jnp.float32),
            jax.ShapeDtypeStruct((ds // CHUNK, n), jnp.float32),
        ],
    )(x, W_enc, b_enc.reshape(1, ds))


def _chunksel_body(cm_ref, vals_ref, cids_ref, *, bt, nck):
    rowios = lax.broadcasted_iota(jnp.int32, (nck, bt), 0)
    kiota = lax.broadcasted_iota(jnp.int32, (bt, KTOP), 1)
    s = cm_ref[...]

    def body(i, carry):
        pm, pi, vals, inds = carry
        eligible = (s < pm[None, :]) | ((s == pm[None, :]) & (rowios > pi[None, :]))
        masked = jnp.where(eligible, s, -jnp.inf)
        m = jnp.max(masked, axis=0)
        idx = jnp.min(jnp.where(masked == m[None, :], rowios, nck), axis=0)
        vals = jnp.where(kiota == i, m[:, None], vals)
        inds = jnp.where(kiota == i, idx[:, None], inds)
        return m, idx, vals, inds

    pm0 = jnp.full((bt,), jnp.inf, jnp.float32)
    pi0 = jnp.full((bt,), -1, jnp.int32)
    v0 = jnp.zeros((bt, KTOP), jnp.float32)
    i0 = jnp.zeros((bt, KTOP), jnp.int32)
    _, _, vals, inds = lax.fori_loop(0, KTOP, body, (pm0, pi0, v0, i0))
    vals_ref[...] = vals
    cids_ref[...] = inds


def _chunksel(cmT, bt, n, tok0):
    nck = cmT.shape[0]
    off_b = tok0 // bt
    return pl.pallas_call(
        functools.partial(_chunksel_body, bt=bt, nck=nck),
        grid=(n // bt,),
        in_specs=[pl.BlockSpec((nck, bt), lambda i: (0, i + off_b))],
        out_specs=[
            pl.BlockSpec((bt, KTOP), lambda i: (i, 0)),
            pl.BlockSpec((bt, KTOP), lambda i: (i, 0)),
        ],
        out_shape=[
            jax.ShapeDtypeStruct((n, KTOP), jnp.float32),
            jax.ShapeDtypeStruct((n, KTOP), jnp.int32),
        ],
    )(cmT)


BIGI = 2 ** 30


def _cand_topk_body(v_ref, i_ref, acts_ref, idx_ref, *, bt):
    s = v_ref[...]
    gi = i_ref[...]
    kiota = lax.broadcasted_iota(jnp.int32, (bt, KTOP), 1)

    def body(i, carry):
        pm, pi, vals, inds = carry
        eligible = (s < pm[:, None]) | ((s == pm[:, None]) & (gi > pi[:, None]))
        masked = jnp.where(eligible, s, -jnp.inf)
        m = jnp.max(masked, axis=1)
        idx = jnp.min(jnp.where(masked == m[:, None], gi, BIGI), axis=1)
        vals = jnp.where(kiota == i, m[:, None], vals)
        inds = jnp.where(kiota == i, idx[:, None], inds)
        return m, idx, vals, inds

    pm0 = jnp.full((bt,), jnp.inf, jnp.float32)
    pi0 = jnp.full((bt,), -1, jnp.int32)
    v0 = jnp.zeros((bt, KTOP), jnp.float32)
    i0 = jnp.zeros((bt, KTOP), jnp.int32)
    _, _, vals, inds = lax.fori_loop(0, KTOP, body, (pm0, pi0, v0, i0))
    acts_ref[...] = vals
    idx_ref[...] = inds


def _cand_topk(cv, ci, bt):
    n, w = cv.shape
    return pl.pallas_call(
        functools.partial(_cand_topk_body, bt=bt),
        grid=(n // bt,),
        in_specs=[pl.BlockSpec((bt, w), lambda i: (i, 0)),
                  pl.BlockSpec((bt, w), lambda i: (i, 0))],
        out_specs=[pl.BlockSpec((bt, KTOP), lambda i: (i, 0)),
                   pl.BlockSpec((bt, KTOP), lambda i: (i, 0))],
        out_shape=[jax.ShapeDtypeStruct((n, KTOP), jnp.float32),
                   jax.ShapeDtypeStruct((n, KTOP), jnp.int32)],
    )(cv, ci)


CANDW = 512
CANDPAD = 544


def _sc_compact(pre2d, cids, cvals, ntot, tok0):
    n = cids.shape[0]
    nck = pre2d.shape[0] // ntot
    info = plsc.get_sparse_core_info()
    nw = info.num_cores * info.num_subcores
    tpw = n // nw
    cc = 32
    mesh = plsc.VectorSubcoreMesh(core_axis_name="c", subcore_axis_name="s")

    @functools.partial(
        pl.kernel,
        mesh=mesh,
        compiler_params=pltpu.CompilerParams(needs_layout_passes=False),
        out_type=[jax.ShapeDtypeStruct((n, CANDW), jnp.float32),
                  jax.ShapeDtypeStruct((n, CANDW), jnp.int32)],
        scratch_types=[
            pltpu.VMEM((cc, KTOP), jnp.int32),
            pltpu.VMEM((cc, KTOP), jnp.float32),
            pltpu.VMEM((16, CHUNK), jnp.float32),
            pltpu.VMEM((16, CHUNK), jnp.float32),
            pltpu.VMEM((16, CHUNK), jnp.float32),
            pltpu.VMEM((16, CHUNK), jnp.float32),
            pltpu.VMEM((CANDPAD,), jnp.float32),
            pltpu.VMEM((CANDPAD,), jnp.int32),
            pltpu.SemaphoreType.DMA,
            pltpu.SemaphoreType.DMA,
        ],
    )
    def comp(pre_hbm, cids_hbm, cvals_hbm, ov_hbm, oi_hbm,
             cid_v, cv_v, a0_v, a1_v, b0_v, b1_v, vb_v, ib_v, sema, semb):
        wid = lax.axis_index("s") * info.num_cores + lax.axis_index("c")
        base = wid * tpw
        lane = lax.iota(jnp.int32, 16)
        candw_v = jnp.full((16,), CANDW, jnp.int32)

        def issue(tl, t, d0, d1, sem):
            c0 = cid_v[tl, pl.ds(0, 16)] + (t + tok0) * nck
            c1 = cid_v[tl, pl.ds(16, 16)] + (t + tok0) * nck
            pltpu.async_copy(pre_hbm.at[c0], d0, sem)
            pltpu.async_copy(pre_hbm.at[c1], d1, sem)

        def drain(d0, d1, sem):
            pltpu.make_async_copy(pre_hbm.at[pl.ds(0, 16)], d0, sem).wait()
            pltpu.make_async_copy(pre_hbm.at[pl.ds(0, 16)], d1, sem).wait()

        def scan(tl, t, d0, d1):
            c0 = cid_v[tl, pl.ds(0, 16)]
            c1 = cid_v[tl, pl.ds(16, 16)]
            t0v = cv_v[tl, pl.ds(16, 16)]
            t0 = t0v[15]
            for r in range(CANDPAD // 16):
                vb_v[pl.ds(16 * r, 16)] = jnp.full((16,), -jnp.inf, jnp.float32)
            offv = jnp.zeros((16,), jnp.int32)
            for j in range(KTOP):
                cvec = c0 if j < 16 else c1
                buf = d0 if j < 16 else d1
                jj = j % 16
                col0 = cvec[jj] * CHUNK
                vs, ms = [], []
                for r in range(CHUNK // 16):
                    v = buf[jj, pl.ds(16 * r, 16)]
                    vs.append(v)
                    ms.append(v >= t0)
                incs = [plsc.cumsum(jnp.where(m, 1, 0)) for m in ms]
                cnts = [plsc.all_reduce_population_count(m) for m in ms]
                b = offv
                for r in range(CHUNK // 16):
                    pos = incs[r] + (b - 1)
                    iv = lane + (col0 + 16 * r)
                    plsc.store_scatter(vb_v, [pos], vs[r], mask=ms[r])
                    plsc.store_scatter(ib_v, [pos], iv, mask=ms[r])
                    b = jnp.minimum(b + cnts[r], candw_v)
                offv = b
            pltpu.sync_copy(vb_v.at[pl.ds(0, CANDW)], ov_hbm.at[t])
            pltpu.sync_copy(ib_v.at[pl.ds(0, CANDW)], oi_hbm.at[t])

        def chunk_body(ci, _):
            cbase = base + ci * cc
            pltpu.sync_copy(cids_hbm.at[pl.ds(cbase, cc)], cid_v)
            pltpu.sync_copy(cvals_hbm.at[pl.ds(cbase, cc)], cv_v)
            issue(0, cbase, a0_v, a1_v, sema)

            def pair_body(p, _):
                tla = 2 * p
                issue(tla + 1, cbase + tla + 1, b0_v, b1_v, semb)
                drain(a0_v, a1_v, sema)
                scan(tla, cbase + tla, a0_v, a1_v)

                @pl.when(p < cc // 2 - 1)
                def _():
                    issue(tla + 2, cbase + tla + 2, a0_v, a1_v, sema)

                drain(b0_v, b1_v, semb)
                scan(tla + 1, cbase + tla + 1, b0_v, b1_v)
                return 0

            lax.fori_loop(0, cc // 2, pair_body, 0)
            return 0

        lax.fori_loop(0, tpw // cc, chunk_body, 0)

    return comp(pre2d, cids, cvals)


def _sc_decode(W_dec_T, acts, idx, b_dec):
    n = acts.shape[0]
    dm = W_dec_T.shape[1]
    nv = dm // 16
    info = plsc.get_sparse_core_info()
    nw = info.num_cores * info.num_subcores
    tpw = n // nw
    cc = 16
    mesh = plsc.VectorSubcoreMesh(core_axis_name="c", subcore_axis_name="s")

    @functools.partial(
        pl.kernel,
        mesh=mesh,
        out_type=jax.ShapeDtypeStruct((n, dm), jnp.float32),
        scratch_types=[
            pltpu.VMEM((cc, KTOP), jnp.int32),
            pltpu.VMEM((cc * KTOP,), jnp.float32),
            pltpu.VMEM((KTOP, 16), jnp.float32),
            pltpu.VMEM((KTOP, dm), jnp.float32),
            pltpu.VMEM((KTOP, dm), jnp.float32),
            pltpu.VMEM((cc, dm), jnp.float32),
            pltpu.VMEM((dm,), jnp.float32),
            pltpu.SemaphoreType.DMA,
            pltpu.SemaphoreType.DMA,
        ],
    )
    def dec(wdt_hbm, acts_hbm, idx_hbm, bd_hbm, out_hbm,
            idx_v, acts_v, asp_v, rowsa_v, rowsb_v, out_v, bias_v, sema, semb):
        wid = lax.axis_index("s") * info.num_cores + lax.axis_index("c")
        base = wid * tpw
        pltpu.sync_copy(bd_hbm, bias_v)

        def compute(tl, rows):
            av0 = acts_v[pl.ds(tl * KTOP, 16)]
            av1 = acts_v[pl.ds(tl * KTOP + 16, 16)]
            for j in range(KTOP):
                a = av0[j] if j < 16 else av1[j - 16]
                asp_v[j, :] = jnp.full((16,), a, jnp.float32)

            nh = nv // 2
            for half in range(2):
                base_v = half * nh

                def jj_body(jj, accs):
                    a0 = asp_v[jj, :]
                    a1 = asp_v[jj + 16, :]
                    return tuple(
                        accs[v]
                        + a0 * rows[jj, pl.ds(16 * (base_v + v), 16)]
                        + a1 * rows[jj + 16, pl.ds(16 * (base_v + v), 16)]
                        for v in range(nh))

                accs0 = tuple(bias_v[pl.ds(16 * (base_v + v), 16)]
                              for v in range(nh))
                accs = lax.fori_loop(0, 16, jj_body, accs0)
                for v in range(nh):
                    out_v[tl, pl.ds(16 * (base_v + v), 16)] = accs[v]

        def drain(dst, sem):
            pltpu.make_async_copy(wdt_hbm.at[pl.ds(0, KTOP)], dst, sem).wait()

        def chunk_body(ci, _):
            cbase = base + ci * cc
            pltpu.sync_copy(idx_hbm.at[pl.ds(cbase, cc)], idx_v)
            pltpu.sync_copy(acts_hbm.at[pl.ds(cbase * KTOP, cc * KTOP)], acts_v)
            pltpu.async_copy(wdt_hbm.at[idx_v.at[0]], rowsa_v, sema)

            def pair_body(p, _):
                tla = 2 * p
                pltpu.async_copy(wdt_hbm.at[idx_v.at[tla + 1]], rowsb_v, semb)
                drain(rowsa_v, sema)
                compute(tla, rowsa_v)

                @pl.when(p < cc // 2 - 1)
                def _():
                    pltpu.async_copy(wdt_hbm.at[idx_v.at[tla + 2]], rowsa_v, sema)

                drain(rowsb_v, semb)
                compute(tla + 1, rowsb_v)
                return 0

            lax.fori_loop(0, cc // 2, pair_body, 0)
            pltpu.sync_copy(out_v, out_hbm.at[pl.ds(cbase, cc)])
            return 0

        lax.fori_loop(0, tpw // cc, chunk_body, 0)

    return dec(W_dec_T, acts.reshape(n * KTOP), idx, b_dec)


def kernel(x, W_enc, b_enc, W_dec, b_dec):
    n, _ = x.shape
    ds = W_enc.shape[0]
    pre, cm = _encode(x, W_enc, b_enc, min(512, n), min(3072, ds))
    nslice = 4
    ns = n // nslice
    pre2d = pre.reshape(n * (ds // CHUNK), CHUNK)
    wdt = jnp.transpose(W_dec)
    parts = []
    for s in range(nslice):
        cvals, cids = _chunksel(cm, 512, ns, s * ns)
        cv, cidx = _sc_compact(pre2d, cids, cvals, n, s * ns)
        acts_s, idx_s = _cand_topk(cv, cidx, 256)
        recon_s = _sc_decode(wdt, acts_s, idx_s, b_dec)
        parts.append((recon_s, acts_s, idx_s))
    recon = jnp.concatenate([p[0] for p in parts], axis=0)
    acts = jnp.concatenate([p[1] for p in parts], axis=0)
    idx = jnp.concatenate([p[2] for p in parts], axis=0)
    return recon, acts, idx
